# baseline (speedup 1.0000x reference)
import functools

import jax
import jax.numpy as jnp
import numpy as np
from jax import lax
from jax.experimental import pallas as pl
from jax.experimental.pallas import tpu as pltpu
from jax.experimental.pallas import tpu_sc as plsc

_N = 4096
_DH = 512
_DL = 32
_K = 15
_A = 1.0 / 0.1 ** 2
_B = float(np.log(2.0))
_EPS = 1e-7
_LN2 = float(np.log(2.0))

_BI1 = 512
_BI2 = 256
_BJ2 = 512

_LN_TAB = np.log(np.maximum(np.arange(4096 + 8, dtype=np.float64),
                            0.5) / 8.0).astype(np.float32)

_NW = 32
_RW = _N // _NW
_NCH = _RW // 16


def _knn_body(x_ref, y_ref, idx_ref, val_ref):
    i = pl.program_id(0)
    x = x_ref[...]
    y = y_ref[...]
    sx = jnp.sum(x * x, axis=1)[:, None]
    sy = jnp.sum(y * y, axis=1)[None, :]
    g = jax.lax.dot_general(x.astype(jnp.bfloat16), y.astype(jnp.bfloat16),
                            (((1,), (1,)), ((), ())),
                            preferred_element_type=jnp.float32)
    sq = sx + sy - 2.0 * g
    iglob = i * _BI1 + jax.lax.broadcasted_iota(jnp.int32, (_BI1, _N), 0)
    jlane = jax.lax.broadcasted_iota(jnp.int32, (_BI1, _N), 1)
    kbits = jax.lax.bitcast_convert_type(jnp.maximum(sq, 0.0), jnp.int32)
    key = jnp.where(jlane == iglob, jnp.int32(0x7FFFFFFF),
                    (kbits & jnp.int32(-4096)) | jlane)
    big = jnp.full((_BI1, 128), 0x7FFFFFFF, jnp.int32)
    slices = [key[:, s * 128:(s + 1) * 128] for s in range(32)]
    ms = []
    for _ in range(4):
        cur = big
        for sl in slices:
            hit = jnp.zeros(sl.shape, jnp.bool_)
            for prev in ms:
                hit = jnp.logical_or(hit, sl == prev)
            cur = jnp.minimum(cur, jnp.where(hit, 0x7FFFFFFF, sl))
        ms.append(cur)
    m1, m2, m3, m4 = ms
    cur = m1
    used = jnp.zeros((_BI1, 128), jnp.int32)
    idxs = []
    vals = []
    kmin = jnp.zeros((_BI1,), jnp.int32)
    for _ in range(_K):
        kmin = jnp.min(cur, axis=1)
        idxs.append(kmin & jnp.int32(0xFFF))
        vals.append(jax.lax.bitcast_convert_type(
            kmin & jnp.int32(-4096), jnp.float32))
        onehot = cur == kmin[:, None]
        nxt = jnp.where(used == 0, m2,
                        jnp.where(used == 1, m3,
                                  jnp.where(used == 2, m4, 0x7FFFFFFF)))
        cur = jnp.where(onehot, nxt, cur)
        used = used + onehot.astype(jnp.int32)
    sigma = jnp.sqrt(jnp.maximum(vals[0], 1e-12))
    idxm = jnp.stack(idxs + [kmin], axis=1)
    valm = jnp.stack(vals + [sigma], axis=1)
    idx_ref[...] = jnp.concatenate(
        [idxm, jnp.zeros((_BI1, 112), jnp.int32)], axis=1)
    val_ref[...] = jnp.concatenate(
        [valm, jnp.zeros((_BI1, 112), jnp.float32)], axis=1)


def _loss_body(ei_ref, ej_ref, sei_ref, sej_ref, out_ref, accL):
    i = pl.program_id(0)
    j = pl.program_id(1)
    nj = pl.num_programs(1)
    ni = pl.num_programs(0)

    @pl.when(jnp.logical_and(i == 0, j == 0))
    def _():
        accL[0, 0] = 0.0

    ei = ei_ref[...]
    ej = ej_ref[...]
    g = jax.lax.dot_general(ei, ej, (((1,), (1,)), ((), ())),
                            preferred_element_type=jnp.float32)
    sq = sei_ref[...] + sej_ref[...] - 2.0 * g
    sql = jnp.maximum(sq, 1e-12)
    u = _A * jnp.exp(_B * jnp.log(sql))
    q = 1.0 / (1.0 + u)
    z = q - _EPS
    lnn_t = -z * (1.0 + z * (0.5 + z * (1.0 / 3.0)))
    accL[0, 0] += jnp.sum(lnn_t)

    @pl.when(j == i // 2)
    def _():
        iglob = i * _BI2 + jax.lax.broadcasted_iota(
            jnp.int32, (_BI2, _BJ2), 0)
        jglob = j * _BJ2 + jax.lax.broadcasted_iota(
            jnp.int32, (_BI2, _BJ2), 1)
        corr = jnp.where(iglob == jglob,
                         jnp.log((1.0 - q) + _EPS) - lnn_t, 0.0)
        accL[0, 0] += jnp.sum(corr)

    @pl.when(jnp.logical_and(i == ni - 1, j == nj - 1))
    def _():
        out_ref[...] = jnp.full((1, 1), accL[0, 0], jnp.float32)


_LNA = float(np.log(_A))
_EMB_SCALE = 4096.0
_TAB_N = 4096


def _edge_body(embp_h, sig_h, se_h, thr_h, tidx_h, lntab_h, valc_h, idxc_h,
               pS_h, pT_h,
               embp_v, sig_v, se_v, thr_v, tidx_v, lntab_v, valt_v, idxt_v,
               S_v, T_v, wscr, lscr, jscr):
    nc = 2
    wid = lax.axis_index("s") * nc + lax.axis_index("c")
    pltpu.sync_copy(embp_h, embp_v)
    pltpu.sync_copy(sig_h, sig_v)
    pltpu.sync_copy(se_h, se_v)
    pltpu.sync_copy(thr_h, thr_v)
    pltpu.sync_copy(tidx_h, tidx_v)
    pltpu.sync_copy(lntab_h, lntab_v)
    pltpu.sync_copy(valc_h.at[wid], valt_v)
    pltpu.sync_copy(idxc_h.at[wid], idxt_v)

    z16 = jnp.zeros((16,), jnp.float32)
    zi16 = jnp.zeros((16,), jnp.int32)

    def _zero(b, _):
        S_v[pl.ds(b * 16, 16)] = z16
        T_v[pl.ds(b * 16, 16)] = z16
        return 0

    lax.fori_loop(0, _N // 16, _zero, 0)
    wscr[_K, :] = z16
    lscr[_K, :] = z16
    jscr[_K, :] = zi16

    iot = lax.iota(jnp.int32, 16)
    lanemask = iot < _K

    def _chunk(c8, _):
        lbase = c8 * 16
        i0g = wid * _RW + lbase
        i_vec = i0g + iot
        sei = se_v[pl.ds(i0g, 16)]
        sigi = sig_v[pl.ds(i0g, 16)]
        inv_sc2 = 1.0 / (_EMB_SCALE * _EMB_SCALE)
        eia = []
        eib = []
        for d in range(16):
            wi = embp_v[d, pl.ds(i0g, 16)]
            eia.append((wi >> 16).astype(jnp.float32))
            eib.append(((wi << 16) >> 16).astype(jnp.float32))
        ownS = z16
        ownT = z16
        for k in range(_K):
            jv = idxt_v[k, pl.ds(lbase, 16)]
            vv = valt_v[k, pl.ds(lbase, 16)]
            sigj = plsc.load_gather(sig_v, [jv])
            sej = plsc.load_gather(se_v, [jv])
            thrj = plsc.load_gather(thr_v, [jv])
            tixj = plsc.load_gather(tidx_v, [jv])
            dot = z16
            for d in range(16):
                wj = plsc.load_gather(embp_v,
                                      [jnp.full((16,), d, jnp.int32), jv])
                aj = (wj >> 16).astype(jnp.float32)
                bj = ((wj << 16) >> 16).astype(jnp.float32)
                dot = dot + eia[d] * aj + eib[d] * bj
            sql = jnp.clip(sei + sej - 2.0 * dot * inv_sc2, 0.25, 511.0)
            sql8 = sql * 8.0
            kidx = sql8.astype(jnp.int32)
            frac = sql8 - kidx.astype(jnp.float32)
            t0 = plsc.load_gather(lntab_v, [kidx])
            t1 = plsc.load_gather(lntab_v, [kidx + 1])
            lnsql = t0 + frac * (t1 - t0)
            u = _A * jnp.exp(_B * lnsql)
            ell = _EPS * u - _LNA - _B * lnsql
            w = jnp.exp(-jnp.maximum(vv, 1e-12) / (sigi * sigj))
            ownS = ownS + w
            ownT = ownT + w * ell
            mut = jnp.logical_or(
                vv < thrj, jnp.logical_and(vv == thrj, i_vec <= tixj))
            nmf = jnp.where(mut, 0.0, 1.0)
            wscr[k, :] = w * nmf
            lscr[k, :] = w * ell * nmf
            jscr[k, :] = jv
        plsc.addupdate(S_v.at[pl.ds(i0g, 16)], ownS)
        plsc.addupdate(T_v.at[pl.ds(i0g, 16)], ownT)
        for r in range(16):
            rv = jnp.full((16,), r, jnp.int32)
            jv2 = plsc.load_gather(jscr, [iot, rv])
            wv2 = plsc.load_gather(wscr, [iot, rv])
            lv2 = plsc.load_gather(lscr, [iot, rv])
            plsc.addupdate_scatter(S_v, [jv2], wv2, mask=lanemask)
            plsc.addupdate_scatter(T_v, [jv2], lv2, mask=lanemask)
        return 0

    lax.fori_loop(0, _NCH, _chunk, 0)
    pltpu.sync_copy(S_v, pS_h.at[wid])
    pltpu.sync_copy(T_v, pT_h.at[wid])


def _combine_body(accl_ref, ps_ref, pt_ref, out_ref):
    S = jnp.sum(ps_ref[...], axis=0)
    T = jnp.sum(pt_ref[...], axis=0)
    psum = jnp.sum(T / jnp.maximum(S, 1e-12))
    out_ref[...] = jnp.full(
        (1, 1), -(accl_ref[0, 0] + psum) / (_N * _N), jnp.float32)


def kernel(embeddings, data):
    idx, val = pl.pallas_call(
        _knn_body,
        grid=(_N // _BI1,),
        in_specs=[
            pl.BlockSpec((_BI1, _DH), lambda i: (i, 0)),
            pl.BlockSpec((_N, _DH), lambda i: (0, 0)),
        ],
        out_specs=[
            pl.BlockSpec((_BI1, 128), lambda i: (i, 0)),
            pl.BlockSpec((_BI1, 128), lambda i: (i, 0)),
        ],
        out_shape=[
            jax.ShapeDtypeStruct((_N, 128), jnp.int32),
            jax.ShapeDtypeStruct((_N, 128), jnp.float32),
        ],
    )(data, data)

    idx15 = idx[:, :_K]
    val15 = val[:, :_K]
    thr = val[:, _K - 1]
    tidx = idx[:, _K - 1]
    sig = val[:, _K]
    idxc = jnp.transpose(
        jnp.concatenate([jnp.transpose(idx15),
                         jnp.zeros((1, _N), jnp.int32)], axis=0)
        .reshape(16, _NW, 128), (1, 0, 2))
    valc = jnp.transpose(
        jnp.concatenate([jnp.transpose(val15),
                         jnp.zeros((1, _N), jnp.float32)], axis=0)
        .reshape(16, _NW, 128), (1, 0, 2))
    se = jnp.sum(embeddings * embeddings, axis=1)
    eq = jnp.clip(jnp.round(embeddings * _EMB_SCALE),
                  -32768, 32767).astype(jnp.int32)
    embp = jnp.transpose(
        (eq[:, 0::2] & 0xFFFF) | (eq[:, 1::2] << 16))
    lntab = jnp.asarray(_LN_TAB)

    edge = pl.kernel(
        _edge_body,
        mesh=plsc.VectorSubcoreMesh(core_axis_name="c", subcore_axis_name="s"),
        compiler_params=pltpu.CompilerParams(needs_layout_passes=False),
        out_type=[
            jax.ShapeDtypeStruct((_NW, _N), jnp.float32),
            jax.ShapeDtypeStruct((_NW, _N), jnp.float32),
        ],
        scratch_types=[
            pltpu.VMEM((16, _N), jnp.int32),
            pltpu.VMEM((_N,), jnp.float32),
            pltpu.VMEM((_N,), jnp.float32),
            pltpu.VMEM((_N,), jnp.float32),
            pltpu.VMEM((_N,), jnp.int32),
            pltpu.VMEM((_TAB_N + 8,), jnp.float32),
            pltpu.VMEM((16, 128), jnp.float32),
            pltpu.VMEM((16, 128), jnp.int32),
            pltpu.VMEM((_N,), jnp.float32),
            pltpu.VMEM((_N,), jnp.float32),
            pltpu.VMEM((16, 16), jnp.float32),
            pltpu.VMEM((16, 16), jnp.float32),
            pltpu.VMEM((16, 16), jnp.int32),
        ],
    )
    pS, pT = edge(embp, sig, se, thr, tidx, lntab, valc, idxc)

    se_col = se[:, None]
    se_row = se[None, :]
    accl = pl.pallas_call(
        _loss_body,
        grid=(_N // _BI2, _N // _BJ2),
        in_specs=[
            pl.BlockSpec((_BI2, _DL), lambda i, j: (i, 0)),
            pl.BlockSpec((_BJ2, _DL), lambda i, j: (j, 0)),
            pl.BlockSpec((_BI2, 1), lambda i, j: (i, 0)),
            pl.BlockSpec((1, _BJ2), lambda i, j: (0, j)),
        ],
        out_specs=pl.BlockSpec((1, 1), lambda i, j: (0, 0)),
        out_shape=jax.ShapeDtypeStruct((1, 1), jnp.float32),
        scratch_shapes=[
            pltpu.SMEM((1, 1), jnp.float32),
        ],
    )(embeddings, embeddings, se_col, se_row)

    out = pl.pallas_call(
        _combine_body,
        grid=(1,),
        in_specs=[
            pl.BlockSpec((1, 1), lambda i: (0, 0)),
            pl.BlockSpec((_NW, _N), lambda i: (0, 0)),
            pl.BlockSpec((_NW, _N), lambda i: (0, 0)),
        ],
        out_specs=pl.BlockSpec((1, 1), lambda i: (0, 0)),
        out_shape=jax.ShapeDtypeStruct((1, 1), jnp.float32),
    )(accl, pS, pT)

    return jnp.reshape(out, ())


# speedup vs baseline: 1.0008x; 1.0008x over previous
import functools

import jax
import jax.numpy as jnp
import numpy as np
from jax import lax
from jax.experimental import pallas as pl
from jax.experimental.pallas import tpu as pltpu
from jax.experimental.pallas import tpu_sc as plsc

_N = 4096
_DH = 512
_DL = 32
_K = 15
_A = 1.0 / 0.1 ** 2
_B = float(np.log(2.0))
_EPS = 1e-7
_LN2 = float(np.log(2.0))

_BI1 = 512
_BI2 = 256
_BJ2 = 512

_LN_TAB = np.log(np.maximum(np.arange(4096 + 8, dtype=np.float64),
                            0.5) / 8.0).astype(np.float32)

_NW = 32
_RW = _N // _NW
_NCH = _RW // 16


def _knn_body(x_ref, y_ref, idx_ref, val_ref):
    i = pl.program_id(0)
    x = x_ref[...]
    y = y_ref[...]
    sx = jnp.sum(x * x, axis=1)[:, None]
    sy = jnp.sum(y * y, axis=1)[None, :]
    g = jax.lax.dot_general(x, y, (((1,), (1,)), ((), ())),
                            preferred_element_type=jnp.float32)
    sq = sx + sy - 2.0 * g
    iglob = i * _BI1 + jax.lax.broadcasted_iota(jnp.int32, (_BI1, _N), 0)
    jlane = jax.lax.broadcasted_iota(jnp.int32, (_BI1, _N), 1)
    kbits = jax.lax.bitcast_convert_type(jnp.maximum(sq, 0.0), jnp.int32)
    key = jnp.where(jlane == iglob, jnp.int32(0x7FFFFFFF),
                    (kbits & jnp.int32(-4096)) | jlane)
    big = jnp.full((_BI1, 128), 0x7FFFFFFF, jnp.int32)
    slices = [key[:, s * 128:(s + 1) * 128] for s in range(32)]
    ms = []
    for _ in range(4):
        cur = big
        for sl in slices:
            hit = jnp.zeros(sl.shape, jnp.bool_)
            for prev in ms:
                hit = jnp.logical_or(hit, sl == prev)
            cur = jnp.minimum(cur, jnp.where(hit, 0x7FFFFFFF, sl))
        ms.append(cur)
    m1, m2, m3, m4 = ms
    cur = m1
    used = jnp.zeros((_BI1, 128), jnp.int32)
    idxs = []
    vals = []
    kmin = jnp.zeros((_BI1,), jnp.int32)
    for _ in range(_K):
        kmin = jnp.min(cur, axis=1)
        idxs.append(kmin & jnp.int32(0xFFF))
        vals.append(jax.lax.bitcast_convert_type(
            kmin & jnp.int32(-4096), jnp.float32))
        onehot = cur == kmin[:, None]
        nxt = jnp.where(used == 0, m2,
                        jnp.where(used == 1, m3,
                                  jnp.where(used == 2, m4, 0x7FFFFFFF)))
        cur = jnp.where(onehot, nxt, cur)
        used = used + onehot.astype(jnp.int32)
    sigma = jnp.sqrt(jnp.maximum(vals[0], 1e-12))
    idxm = jnp.stack(idxs + [kmin], axis=1)
    valm = jnp.stack(vals + [sigma], axis=1)
    idx_ref[...] = jnp.concatenate(
        [idxm, jnp.zeros((_BI1, 112), jnp.int32)], axis=1)
    val_ref[...] = jnp.concatenate(
        [valm, jnp.zeros((_BI1, 112), jnp.float32)], axis=1)


def _loss_body(ei_ref, ej_ref, sei_ref, sej_ref, out_ref, accL):
    i = pl.program_id(0)
    j = pl.program_id(1)
    nj = pl.num_programs(1)
    ni = pl.num_programs(0)

    @pl.when(jnp.logical_and(i == 0, j == 0))
    def _():
        accL[0, 0] = 0.0

    ei = ei_ref[...]
    ej = ej_ref[...]
    g = jax.lax.dot_general(ei, ej, (((1,), (1,)), ((), ())),
                            preferred_element_type=jnp.float32)
    sq = sei_ref[...] + sej_ref[...] - 2.0 * g
    sql = jnp.maximum(sq, 1e-12)
    u = _A * jnp.exp(_B * jnp.log(sql))
    q = 1.0 / (1.0 + u)
    z = q - _EPS
    lnn_t = -z * (1.0 + z * (0.5 + z * (1.0 / 3.0)))
    accL[0, 0] += jnp.sum(lnn_t)

    @pl.when(j == i // 2)
    def _():
        iglob = i * _BI2 + jax.lax.broadcasted_iota(
            jnp.int32, (_BI2, _BJ2), 0)
        jglob = j * _BJ2 + jax.lax.broadcasted_iota(
            jnp.int32, (_BI2, _BJ2), 1)
        corr = jnp.where(iglob == jglob,
                         jnp.log((1.0 - q) + _EPS) - lnn_t, 0.0)
        accL[0, 0] += jnp.sum(corr)

    @pl.when(jnp.logical_and(i == ni - 1, j == nj - 1))
    def _():
        out_ref[...] = jnp.full((1, 1), accL[0, 0], jnp.float32)


_LNA = float(np.log(_A))
_EMB_SCALE = 4096.0
_TAB_N = 4096


def _edge_body(embp_h, sig_h, se_h, thr_h, tidx_h, lntab_h, valc_h, idxc_h,
               pS_h, pT_h,
               embp_v, sig_v, se_v, thr_v, tidx_v, lntab_v, valt_v, idxt_v,
               S_v, T_v, wscr, lscr, jscr):
    nc = 2
    wid = lax.axis_index("s") * nc + lax.axis_index("c")
    pltpu.sync_copy(embp_h, embp_v)
    pltpu.sync_copy(sig_h, sig_v)
    pltpu.sync_copy(se_h, se_v)
    pltpu.sync_copy(thr_h, thr_v)
    pltpu.sync_copy(tidx_h, tidx_v)
    pltpu.sync_copy(lntab_h, lntab_v)
    pltpu.sync_copy(valc_h.at[wid], valt_v)
    pltpu.sync_copy(idxc_h.at[wid], idxt_v)

    z16 = jnp.zeros((16,), jnp.float32)
    zi16 = jnp.zeros((16,), jnp.int32)

    def _zero(b, _):
        S_v[pl.ds(b * 16, 16)] = z16
        T_v[pl.ds(b * 16, 16)] = z16
        return 0

    lax.fori_loop(0, _N // 16, _zero, 0)
    wscr[_K, :] = z16
    lscr[_K, :] = z16
    jscr[_K, :] = zi16

    iot = lax.iota(jnp.int32, 16)
    lanemask = iot < _K

    def _chunk(c8, _):
        lbase = c8 * 16
        i0g = wid * _RW + lbase
        i_vec = i0g + iot
        sei = se_v[pl.ds(i0g, 16)]
        sigi = sig_v[pl.ds(i0g, 16)]
        inv_sc2 = 1.0 / (_EMB_SCALE * _EMB_SCALE)
        eia = []
        eib = []
        for d in range(16):
            wi = embp_v[d, pl.ds(i0g, 16)]
            eia.append((wi >> 16).astype(jnp.float32))
            eib.append(((wi << 16) >> 16).astype(jnp.float32))
        ownS = z16
        ownT = z16
        for k in range(_K):
            jv = idxt_v[k, pl.ds(lbase, 16)]
            vv = valt_v[k, pl.ds(lbase, 16)]
            sigj = plsc.load_gather(sig_v, [jv])
            sej = plsc.load_gather(se_v, [jv])
            thrj = plsc.load_gather(thr_v, [jv])
            tixj = plsc.load_gather(tidx_v, [jv])
            dot = z16
            for d in range(16):
                wj = plsc.load_gather(embp_v,
                                      [jnp.full((16,), d, jnp.int32), jv])
                aj = (wj >> 16).astype(jnp.float32)
                bj = ((wj << 16) >> 16).astype(jnp.float32)
                dot = dot + eia[d] * aj + eib[d] * bj
            sql = jnp.clip(sei + sej - 2.0 * dot * inv_sc2, 0.25, 511.0)
            sql8 = sql * 8.0
            kidx = sql8.astype(jnp.int32)
            frac = sql8 - kidx.astype(jnp.float32)
            t0 = plsc.load_gather(lntab_v, [kidx])
            t1 = plsc.load_gather(lntab_v, [kidx + 1])
            lnsql = t0 + frac * (t1 - t0)
            u = _A * jnp.exp(_B * lnsql)
            ell = _EPS * u - _LNA - _B * lnsql
            w = jnp.exp(-jnp.maximum(vv, 1e-12) / (sigi * sigj))
            ownS = ownS + w
            ownT = ownT + w * ell
            mut = jnp.logical_or(
                vv < thrj, jnp.logical_and(vv == thrj, i_vec <= tixj))
            nmf = jnp.where(mut, 0.0, 1.0)
            wscr[k, :] = w * nmf
            lscr[k, :] = w * ell * nmf
            jscr[k, :] = jv
        plsc.addupdate(S_v.at[pl.ds(i0g, 16)], ownS)
        plsc.addupdate(T_v.at[pl.ds(i0g, 16)], ownT)
        for r in range(16):
            rv = jnp.full((16,), r, jnp.int32)
            jv2 = plsc.load_gather(jscr, [iot, rv])
            wv2 = plsc.load_gather(wscr, [iot, rv])
            lv2 = plsc.load_gather(lscr, [iot, rv])
            plsc.addupdate_scatter(S_v, [jv2], wv2, mask=lanemask)
            plsc.addupdate_scatter(T_v, [jv2], lv2, mask=lanemask)
        return 0

    lax.fori_loop(0, _NCH, _chunk, 0)
    pltpu.sync_copy(S_v, pS_h.at[wid])
    pltpu.sync_copy(T_v, pT_h.at[wid])


def _combine_body(accl_ref, ps_ref, pt_ref, out_ref):
    S = jnp.sum(ps_ref[...], axis=0)
    T = jnp.sum(pt_ref[...], axis=0)
    psum = jnp.sum(T / jnp.maximum(S, 1e-12))
    out_ref[...] = jnp.full(
        (1, 1), -(accl_ref[0, 0] + psum) / (_N * _N), jnp.float32)


def kernel(embeddings, data):
    idx, val = pl.pallas_call(
        _knn_body,
        grid=(_N // _BI1,),
        in_specs=[
            pl.BlockSpec((_BI1, _DH), lambda i: (i, 0)),
            pl.BlockSpec((_N, _DH), lambda i: (0, 0)),
        ],
        out_specs=[
            pl.BlockSpec((_BI1, 128), lambda i: (i, 0)),
            pl.BlockSpec((_BI1, 128), lambda i: (i, 0)),
        ],
        out_shape=[
            jax.ShapeDtypeStruct((_N, 128), jnp.int32),
            jax.ShapeDtypeStruct((_N, 128), jnp.float32),
        ],
    )(data, data)

    idx15 = idx[:, :_K]
    val15 = val[:, :_K]
    thr = val[:, _K - 1]
    tidx = idx[:, _K - 1]
    sig = val[:, _K]
    idxc = jnp.transpose(
        jnp.concatenate([jnp.transpose(idx15),
                         jnp.zeros((1, _N), jnp.int32)], axis=0)
        .reshape(16, _NW, 128), (1, 0, 2))
    valc = jnp.transpose(
        jnp.concatenate([jnp.transpose(val15),
                         jnp.zeros((1, _N), jnp.float32)], axis=0)
        .reshape(16, _NW, 128), (1, 0, 2))
    se = jnp.sum(embeddings * embeddings, axis=1)
    eq = jnp.clip(jnp.round(embeddings * _EMB_SCALE),
                  -32768, 32767).astype(jnp.int32)
    embp = jnp.transpose(
        (eq[:, 0::2] & 0xFFFF) | (eq[:, 1::2] << 16))
    lntab = jnp.asarray(_LN_TAB)

    edge = pl.kernel(
        _edge_body,
        mesh=plsc.VectorSubcoreMesh(core_axis_name="c", subcore_axis_name="s"),
        compiler_params=pltpu.CompilerParams(needs_layout_passes=False),
        out_type=[
            jax.ShapeDtypeStruct((_NW, _N), jnp.float32),
            jax.ShapeDtypeStruct((_NW, _N), jnp.float32),
        ],
        scratch_types=[
            pltpu.VMEM((16, _N), jnp.int32),
            pltpu.VMEM((_N,), jnp.float32),
            pltpu.VMEM((_N,), jnp.float32),
            pltpu.VMEM((_N,), jnp.float32),
            pltpu.VMEM((_N,), jnp.int32),
            pltpu.VMEM((_TAB_N + 8,), jnp.float32),
            pltpu.VMEM((16, 128), jnp.float32),
            pltpu.VMEM((16, 128), jnp.int32),
            pltpu.VMEM((_N,), jnp.float32),
            pltpu.VMEM((_N,), jnp.float32),
            pltpu.VMEM((16, 16), jnp.float32),
            pltpu.VMEM((16, 16), jnp.float32),
            pltpu.VMEM((16, 16), jnp.int32),
        ],
    )
    pS, pT = edge(embp, sig, se, thr, tidx, lntab, valc, idxc)

    se_col = se[:, None]
    se_row = se[None, :]
    accl = pl.pallas_call(
        _loss_body,
        grid=(_N // _BI2, _N // _BJ2),
        in_specs=[
            pl.BlockSpec((_BI2, _DL), lambda i, j: (i, 0)),
            pl.BlockSpec((_BJ2, _DL), lambda i, j: (j, 0)),
            pl.BlockSpec((_BI2, 1), lambda i, j: (i, 0)),
            pl.BlockSpec((1, _BJ2), lambda i, j: (0, j)),
        ],
        out_specs=pl.BlockSpec((1, 1), lambda i, j: (0, 0)),
        out_shape=jax.ShapeDtypeStruct((1, 1), jnp.float32),
        scratch_shapes=[
            pltpu.SMEM((1, 1), jnp.float32),
        ],
    )(embeddings, embeddings, se_col, se_row)

    out = pl.pallas_call(
        _combine_body,
        grid=(1,),
        in_specs=[
            pl.BlockSpec((1, 1), lambda i: (0, 0)),
            pl.BlockSpec((_NW, _N), lambda i: (0, 0)),
            pl.BlockSpec((_NW, _N), lambda i: (0, 0)),
        ],
        out_specs=pl.BlockSpec((1, 1), lambda i: (0, 0)),
        out_shape=jax.ShapeDtypeStruct((1, 1), jnp.float32),
    )(accl, pS, pT)

    return jnp.reshape(out, ())


# speedup vs baseline: 1.1340x; 1.1331x over previous
import functools

import jax
import jax.numpy as jnp
import numpy as np
from jax import lax
from jax.experimental import pallas as pl
from jax.experimental.pallas import tpu as pltpu
from jax.experimental.pallas import tpu_sc as plsc

_N = 4096
_DH = 512
_DL = 32
_K = 15
_A = 1.0 / 0.1 ** 2
_B = float(np.log(2.0))
_EPS = 1e-7
_LN2 = float(np.log(2.0))

_BI1 = 512
_BI2 = 256
_BJ2 = 512
_BB2 = 256

_LN_TAB = np.log(np.maximum(np.arange(4096 + 8, dtype=np.float64),
                            0.5) / 8.0).astype(np.float32)

_NW = 32
_RW = _N // _NW
_NCH = _RW // 16


def _knn_body(x_ref, y_ref, xb_ref, yb_ref, idx_ref, val_ref):
    i = pl.program_id(0)
    x = x_ref[...]
    y = y_ref[...]
    sx = jnp.sum(x * x, axis=1)[:, None]
    sy = jnp.sum(y * y, axis=1)[None, :]
    g = jax.lax.dot_general(xb_ref[...], yb_ref[...],
                            (((1,), (1,)), ((), ())),
                            preferred_element_type=jnp.float32)
    sq = sx + sy - 2.0 * g
    iglob = i * _BI1 + jax.lax.broadcasted_iota(jnp.int32, (_BI1, _N), 0)
    jlane = jax.lax.broadcasted_iota(jnp.int32, (_BI1, _N), 1)
    kbits = jax.lax.bitcast_convert_type(jnp.maximum(sq, 0.0), jnp.int32)
    key = jnp.where(jlane == iglob, jnp.int32(0x7FFFFFFF),
                    (kbits & jnp.int32(-4096)) | jlane)
    big = jnp.full((_BI1, 128), 0x7FFFFFFF, jnp.int32)
    slices = [key[:, s * 128:(s + 1) * 128] for s in range(32)]
    ms = []
    for _ in range(2):
        cur = big
        for sl in slices:
            hit = jnp.zeros(sl.shape, jnp.bool_)
            for prev in ms:
                hit = jnp.logical_or(hit, sl == prev)
            cur = jnp.minimum(cur, jnp.where(hit, 0x7FFFFFFF, sl))
        ms.append(cur)
    m1, m2 = ms
    cur = m1
    used = jnp.zeros((_BI1, 128), jnp.int32)
    idxs = []
    vals = []
    kmin = jnp.zeros((_BI1,), jnp.int32)
    for _ in range(_K):
        kmin = jnp.min(cur, axis=1)
        idxs.append(kmin & jnp.int32(0xFFF))
        vals.append(jax.lax.bitcast_convert_type(
            kmin & jnp.int32(-4096), jnp.float32))
        onehot = cur == kmin[:, None]
        nxt = jnp.where(used == 0, m2, 0x7FFFFFFF)
        cur = jnp.where(onehot, nxt, cur)
        used = used + onehot.astype(jnp.int32)
    sigma = jnp.sqrt(jnp.maximum(vals[0], 1e-12))
    idxm = jnp.stack(idxs + [kmin], axis=1)
    valm = jnp.stack(vals + [sigma], axis=1)
    idx_ref[...] = jnp.concatenate(
        [idxm, jnp.zeros((_BI1, 112), jnp.int32)], axis=1)
    val_ref[...] = jnp.concatenate(
        [valm, jnp.zeros((_BI1, 112), jnp.float32)], axis=1)


def _loss_body(ei_ref, ej_ref, sei_ref, sej_ref, out_ref, accL):
    i = pl.program_id(0)
    t = pl.program_id(1)
    nb = pl.num_programs(0)
    nt = pl.num_programs(1)

    @pl.when(jnp.logical_and(i == 0, t == 0))
    def _():
        accL[0, 0] = 0.0

    @pl.when(jnp.logical_or(t < nt - 1, i < nb // 2))
    def _():
        ei = ei_ref[...]
        ej = ej_ref[...]
        g = jax.lax.dot_general(ei, ej, (((1,), (1,)), ((), ())),
                                preferred_element_type=jnp.float32)
        sq = sei_ref[...] + sej_ref[...] - 2.0 * g
        sql = jnp.maximum(sq, 1e-12)
        u = _A * jnp.exp(_B * jnp.log(sql))
        q = 1.0 / (1.0 + u)
        lnn = jnp.log((1.0 - q) + _EPS)
        wgt = jnp.where(t == 0, 1.0, 2.0)
        accL[0, 0] += wgt * jnp.sum(lnn)

    @pl.when(jnp.logical_and(i == nb - 1, t == nt - 1))
    def _():
        out_ref[...] = jnp.full((1, 1), accL[0, 0], jnp.float32)


_LNA = float(np.log(_A))
_EMB_SCALE = 4096.0
_TAB_N = 4096


def _edge_body(embp_h, sig_h, se_h, thr_h, tidx_h, lntab_h, valc_h, idxc_h,
               pS_h, pT_h,
               embp_v, sig_v, se_v, thr_v, tidx_v, lntab_v, valt_v, idxt_v,
               S_v, T_v, wscr, lscr, jscr):
    nc = 2
    wid = lax.axis_index("s") * nc + lax.axis_index("c")
    pltpu.sync_copy(embp_h, embp_v)
    pltpu.sync_copy(sig_h, sig_v)
    pltpu.sync_copy(se_h, se_v)
    pltpu.sync_copy(thr_h, thr_v)
    pltpu.sync_copy(tidx_h, tidx_v)
    pltpu.sync_copy(lntab_h, lntab_v)
    pltpu.sync_copy(valc_h.at[wid], valt_v)
    pltpu.sync_copy(idxc_h.at[wid], idxt_v)

    z16 = jnp.zeros((16,), jnp.float32)
    zi16 = jnp.zeros((16,), jnp.int32)

    def _zero(b, _):
        S_v[pl.ds(b * 16, 16)] = z16
        T_v[pl.ds(b * 16, 16)] = z16
        return 0

    lax.fori_loop(0, _N // 16, _zero, 0)
    wscr[_K, :] = z16
    lscr[_K, :] = z16
    jscr[_K, :] = zi16

    iot = lax.iota(jnp.int32, 16)
    lanemask = iot < _K

    def _chunk(c8, _):
        lbase = c8 * 16
        i0g = wid * _RW + lbase
        i_vec = i0g + iot
        sei = se_v[pl.ds(i0g, 16)]
        sigi = sig_v[pl.ds(i0g, 16)]
        inv_sc2 = 1.0 / (_EMB_SCALE * _EMB_SCALE)
        eia = []
        eib = []
        for d in range(16):
            wi = embp_v[d, pl.ds(i0g, 16)]
            eia.append((wi >> 16).astype(jnp.float32))
            eib.append(((wi << 16) >> 16).astype(jnp.float32))
        ownS = z16
        ownT = z16
        for k in range(_K):
            jv = idxt_v[k, pl.ds(lbase, 16)]
            vv = valt_v[k, pl.ds(lbase, 16)]
            sigj = plsc.load_gather(sig_v, [jv])
            sej = plsc.load_gather(se_v, [jv])
            thrj = plsc.load_gather(thr_v, [jv])
            tixj = plsc.load_gather(tidx_v, [jv])
            dot = z16
            for d in range(16):
                wj = plsc.load_gather(embp_v,
                                      [jnp.full((16,), d, jnp.int32), jv])
                aj = (wj >> 16).astype(jnp.float32)
                bj = ((wj << 16) >> 16).astype(jnp.float32)
                dot = dot + eia[d] * aj + eib[d] * bj
            sql = jnp.clip(sei + sej - 2.0 * dot * inv_sc2, 0.25, 511.0)
            sql8 = sql * 8.0
            kidx = sql8.astype(jnp.int32)
            frac = sql8 - kidx.astype(jnp.float32)
            t0 = plsc.load_gather(lntab_v, [kidx])
            t1 = plsc.load_gather(lntab_v, [kidx + 1])
            lnsql = t0 + frac * (t1 - t0)
            u = _A * jnp.exp(_B * lnsql)
            ell = _EPS * u - _LNA - _B * lnsql
            w = jnp.exp(-jnp.maximum(vv, 1e-12) / (sigi * sigj))
            ownS = ownS + w
            ownT = ownT + w * ell
            mut = jnp.logical_or(
                vv < thrj, jnp.logical_and(vv == thrj, i_vec <= tixj))
            nmf = jnp.where(mut, 0.0, 1.0)
            wscr[k, :] = w * nmf
            lscr[k, :] = w * ell * nmf
            jscr[k, :] = jv
        plsc.addupdate(S_v.at[pl.ds(i0g, 16)], ownS)
        plsc.addupdate(T_v.at[pl.ds(i0g, 16)], ownT)
        for r in range(16):
            rv = jnp.full((16,), r, jnp.int32)
            jv2 = plsc.load_gather(jscr, [iot, rv])
            wv2 = plsc.load_gather(wscr, [iot, rv])
            lv2 = plsc.load_gather(lscr, [iot, rv])
            plsc.addupdate_scatter(S_v, [jv2], wv2, mask=lanemask)
            plsc.addupdate_scatter(T_v, [jv2], lv2, mask=lanemask)
        return 0

    lax.fori_loop(0, _NCH, _chunk, 0)
    pltpu.sync_copy(S_v, pS_h.at[wid])
    pltpu.sync_copy(T_v, pT_h.at[wid])


def _combine_body(accl_ref, ps_ref, pt_ref, out_ref):
    S = jnp.sum(ps_ref[...], axis=0)
    T = jnp.sum(pt_ref[...], axis=0)
    psum = jnp.sum(T / jnp.maximum(S, 1e-12))
    out_ref[...] = jnp.full(
        (1, 1), -(accl_ref[0, 0] + psum) / (_N * _N), jnp.float32)


def kernel(embeddings, data):
    data_bf = data.astype(jnp.bfloat16)
    idx, val = pl.pallas_call(
        _knn_body,
        grid=(_N // _BI1,),
        in_specs=[
            pl.BlockSpec((_BI1, _DH), lambda i: (i, 0)),
            pl.BlockSpec((_N, _DH), lambda i: (0, 0)),
            pl.BlockSpec((_BI1, _DH), lambda i: (i, 0)),
            pl.BlockSpec((_N, _DH), lambda i: (0, 0)),
        ],
        out_specs=[
            pl.BlockSpec((_BI1, 128), lambda i: (i, 0)),
            pl.BlockSpec((_BI1, 128), lambda i: (i, 0)),
        ],
        out_shape=[
            jax.ShapeDtypeStruct((_N, 128), jnp.int32),
            jax.ShapeDtypeStruct((_N, 128), jnp.float32),
        ],
    )(data, data, data_bf, data_bf)

    idx15 = idx[:, :_K]
    val15 = val[:, :_K]
    thr = val[:, _K - 1]
    tidx = idx[:, _K - 1]
    sig = val[:, _K]
    idxc = jnp.transpose(
        jnp.concatenate([jnp.transpose(idx15),
                         jnp.zeros((1, _N), jnp.int32)], axis=0)
        .reshape(16, _NW, 128), (1, 0, 2))
    valc = jnp.transpose(
        jnp.concatenate([jnp.transpose(val15),
                         jnp.zeros((1, _N), jnp.float32)], axis=0)
        .reshape(16, _NW, 128), (1, 0, 2))
    se = jnp.sum(embeddings * embeddings, axis=1)
    eq = jnp.clip(jnp.round(embeddings * _EMB_SCALE),
                  -32768, 32767).astype(jnp.int32)
    embp = jnp.transpose(
        (eq[:, 0::2] & 0xFFFF) | (eq[:, 1::2] << 16))
    lntab = jnp.asarray(_LN_TAB)

    edge = pl.kernel(
        _edge_body,
        mesh=plsc.VectorSubcoreMesh(core_axis_name="c", subcore_axis_name="s"),
        compiler_params=pltpu.CompilerParams(needs_layout_passes=False),
        out_type=[
            jax.ShapeDtypeStruct((_NW, _N), jnp.float32),
            jax.ShapeDtypeStruct((_NW, _N), jnp.float32),
        ],
        scratch_types=[
            pltpu.VMEM((16, _N), jnp.int32),
            pltpu.VMEM((_N,), jnp.float32),
            pltpu.VMEM((_N,), jnp.float32),
            pltpu.VMEM((_N,), jnp.float32),
            pltpu.VMEM((_N,), jnp.int32),
            pltpu.VMEM((_TAB_N + 8,), jnp.float32),
            pltpu.VMEM((16, 128), jnp.float32),
            pltpu.VMEM((16, 128), jnp.int32),
            pltpu.VMEM((_N,), jnp.float32),
            pltpu.VMEM((_N,), jnp.float32),
            pltpu.VMEM((16, 16), jnp.float32),
            pltpu.VMEM((16, 16), jnp.float32),
            pltpu.VMEM((16, 16), jnp.int32),
        ],
    )
    pS, pT = edge(embp, sig, se, thr, tidx, lntab, valc, idxc)

    se_col = se[:, None]
    se_row = se[None, :]
    nb = _N // _BB2
    accl = pl.pallas_call(
        _loss_body,
        grid=(nb, nb // 2 + 1),
        in_specs=[
            pl.BlockSpec((_BB2, _DL), lambda i, t: (i, 0)),
            pl.BlockSpec((_BB2, _DL), lambda i, t: ((i + t) % (_N // _BB2), 0)),
            pl.BlockSpec((_BB2, 1), lambda i, t: (i, 0)),
            pl.BlockSpec((1, _BB2), lambda i, t: (0, (i + t) % (_N // _BB2))),
        ],
        out_specs=pl.BlockSpec((1, 1), lambda i, t: (0, 0)),
        out_shape=jax.ShapeDtypeStruct((1, 1), jnp.float32),
        scratch_shapes=[
            pltpu.SMEM((1, 1), jnp.float32),
        ],
    )(embeddings, embeddings, se_col, se_row)

    out = pl.pallas_call(
        _combine_body,
        grid=(1,),
        in_specs=[
            pl.BlockSpec((1, 1), lambda i: (0, 0)),
            pl.BlockSpec((_NW, _N), lambda i: (0, 0)),
            pl.BlockSpec((_NW, _N), lambda i: (0, 0)),
        ],
        out_specs=pl.BlockSpec((1, 1), lambda i: (0, 0)),
        out_shape=jax.ShapeDtypeStruct((1, 1), jnp.float32),
    )(accl, pS, pT)

    return jnp.reshape(out, ())


# speedup vs baseline: 1.5627x; 1.3781x over previous
import functools

import jax
import jax.numpy as jnp
import numpy as np
from jax import lax
from jax.experimental import pallas as pl
from jax.experimental.pallas import tpu as pltpu
from jax.experimental.pallas import tpu_sc as plsc

_N = 4096
_DH = 512
_DL = 32
_K = 15
_A = 1.0 / 0.1 ** 2
_B = float(np.log(2.0))
_EPS = 1e-7
_LN2 = float(np.log(2.0))

_BI1 = 512
_BI2 = 256
_BJ2 = 512
_BB2 = 512

_LN_TAB = np.log(np.maximum(np.arange(4096 + 8, dtype=np.float64),
                            0.5) / 8.0).astype(np.float32)

_NW = 32
_RW = _N // _NW
_NCH = _RW // 16


def _knn_body(x_ref, y_ref, xb_ref, yb_ref, idx_ref, val_ref):
    i = pl.program_id(0)
    x = x_ref[...]
    y = y_ref[...]
    sx = jnp.sum(x * x, axis=1)[:, None]
    sy = jnp.sum(y * y, axis=1)[None, :]
    g = jax.lax.dot_general(xb_ref[...], yb_ref[...],
                            (((1,), (1,)), ((), ())),
                            preferred_element_type=jnp.float32)
    sq = sx + sy - 2.0 * g
    iglob = i * _BI1 + jax.lax.broadcasted_iota(jnp.int32, (_BI1, _N), 0)
    jlane = jax.lax.broadcasted_iota(jnp.int32, (_BI1, _N), 1)
    kbits = jax.lax.bitcast_convert_type(jnp.maximum(sq, 0.0), jnp.int32)
    key = jnp.where(jlane == iglob, jnp.int32(0x7FFFFFFF),
                    (kbits & jnp.int32(-4096)) | jlane)
    big = jnp.full((_BI1, 128), 0x7FFFFFFF, jnp.int32)
    slices = [key[:, s * 128:(s + 1) * 128] for s in range(32)]
    ms = []
    for _ in range(2):
        cur = big
        for sl in slices:
            hit = jnp.zeros(sl.shape, jnp.bool_)
            for prev in ms:
                hit = jnp.logical_or(hit, sl == prev)
            cur = jnp.minimum(cur, jnp.where(hit, 0x7FFFFFFF, sl))
        ms.append(cur)
    m1, m2 = ms
    cur = m1
    used = jnp.zeros((_BI1, 128), jnp.int32)
    idxs = []
    vals = []
    kmin = jnp.zeros((_BI1,), jnp.int32)
    for _ in range(_K):
        kmin = jnp.min(cur, axis=1)
        idxs.append(kmin & jnp.int32(0xFFF))
        vals.append(jax.lax.bitcast_convert_type(
            kmin & jnp.int32(-4096), jnp.float32))
        onehot = cur == kmin[:, None]
        nxt = jnp.where(used == 0, m2, 0x7FFFFFFF)
        cur = jnp.where(onehot, nxt, cur)
        used = used + onehot.astype(jnp.int32)
    sigma = jnp.sqrt(jnp.maximum(vals[0], 1e-12))
    idxm = jnp.stack(idxs + [kmin], axis=1)
    valm = jnp.stack(vals + [sigma], axis=1)
    idx_ref[...] = jnp.concatenate(
        [idxm, jnp.zeros((_BI1, 112), jnp.int32)], axis=1)
    val_ref[...] = jnp.concatenate(
        [valm, jnp.zeros((_BI1, 112), jnp.float32)], axis=1)


def _loss_body(ei_ref, ej_ref, sei_ref, sej_ref, out_ref, accL):
    i = pl.program_id(0)
    t = pl.program_id(1)
    nb = pl.num_programs(0)
    nt = pl.num_programs(1)

    @pl.when(jnp.logical_and(i == 0, t == 0))
    def _():
        accL[0, 0] = 0.0

    @pl.when(jnp.logical_or(t < nt - 1, i < nb // 2))
    def _():
        ei = ei_ref[...]
        ej = ej_ref[...]
        g = jax.lax.dot_general(ei, ej, (((1,), (1,)), ((), ())),
                                preferred_element_type=jnp.float32)
        sq = sei_ref[...] + sej_ref[...] - 2.0 * g
        sql = jnp.maximum(sq, 1e-12)
        u = _A * jnp.exp(_B * jnp.log(sql))
        q = 1.0 / (1.0 + u)
        lnn = jnp.log((1.0 - q) + _EPS)
        wgt = jnp.where(t == 0, 1.0, 2.0)
        accL[0, 0] += wgt * jnp.sum(lnn)

    @pl.when(jnp.logical_and(i == nb - 1, t == nt - 1))
    def _():
        out_ref[...] = jnp.full((1, 1), accL[0, 0], jnp.float32)


_LNA = float(np.log(_A))
_EMB_SCALE = 4096.0
_TAB_N = 4096


def _edge_body(embp_h, sig_h, se_h, thr_h, tidx_h, lntab_h, valc_h, idxc_h,
               pS_h, pT_h,
               embp_v, sig_v, se_v, thr_v, tidx_v, lntab_v, valt_v, idxt_v,
               S_v, T_v, wscr, lscr, jscr):
    nc = 2
    wid = lax.axis_index("s") * nc + lax.axis_index("c")
    pltpu.sync_copy(embp_h, embp_v)
    pltpu.sync_copy(sig_h, sig_v)
    pltpu.sync_copy(se_h, se_v)
    pltpu.sync_copy(thr_h, thr_v)
    pltpu.sync_copy(tidx_h, tidx_v)
    pltpu.sync_copy(lntab_h, lntab_v)
    pltpu.sync_copy(valc_h.at[wid], valt_v)
    pltpu.sync_copy(idxc_h.at[wid], idxt_v)

    z16 = jnp.zeros((16,), jnp.float32)
    zi16 = jnp.zeros((16,), jnp.int32)

    def _zero(b, _):
        S_v[pl.ds(b * 16, 16)] = z16
        T_v[pl.ds(b * 16, 16)] = z16
        return 0

    lax.fori_loop(0, _N // 16, _zero, 0)
    wscr[_K, :] = z16
    lscr[_K, :] = z16
    jscr[_K, :] = zi16

    iot = lax.iota(jnp.int32, 16)
    lanemask = iot < _K

    def _chunk(c8, _):
        lbase = c8 * 16
        i0g = wid * _RW + lbase
        i_vec = i0g + iot
        sei = se_v[pl.ds(i0g, 16)]
        sigi = sig_v[pl.ds(i0g, 16)]
        inv_sc2 = 1.0 / (_EMB_SCALE * _EMB_SCALE)
        eia = []
        eib = []
        for d in range(16):
            wi = embp_v[d, pl.ds(i0g, 16)]
            eia.append((wi >> 16).astype(jnp.float32))
            eib.append(((wi << 16) >> 16).astype(jnp.float32))
        ownS = z16
        ownT = z16
        for k in range(_K):
            jv = idxt_v[k, pl.ds(lbase, 16)]
            vv = valt_v[k, pl.ds(lbase, 16)]
            sigj = plsc.load_gather(sig_v, [jv])
            sej = plsc.load_gather(se_v, [jv])
            thrj = plsc.load_gather(thr_v, [jv])
            tixj = plsc.load_gather(tidx_v, [jv])
            dot = z16
            for d in range(16):
                wj = plsc.load_gather(embp_v,
                                      [jnp.full((16,), d, jnp.int32), jv])
                aj = (wj >> 16).astype(jnp.float32)
                bj = ((wj << 16) >> 16).astype(jnp.float32)
                dot = dot + eia[d] * aj + eib[d] * bj
            sql = jnp.clip(sei + sej - 2.0 * dot * inv_sc2, 0.25, 511.0)
            sql8 = sql * 8.0
            kidx = sql8.astype(jnp.int32)
            frac = sql8 - kidx.astype(jnp.float32)
            t0 = plsc.load_gather(lntab_v, [kidx])
            t1 = plsc.load_gather(lntab_v, [kidx + 1])
            lnsql = t0 + frac * (t1 - t0)
            u = _A * jnp.exp(_B * lnsql)
            ell = _EPS * u - _LNA - _B * lnsql
            w = jnp.exp(-jnp.maximum(vv, 1e-12) / (sigi * sigj))
            ownS = ownS + w
            ownT = ownT + w * ell
            mut = jnp.logical_or(
                vv < thrj, jnp.logical_and(vv == thrj, i_vec <= tixj))
            nmf = jnp.where(mut, 0.0, 1.0)
            wscr[k, :] = w * nmf
            lscr[k, :] = w * ell * nmf
            jscr[k, :] = jv
        plsc.addupdate(S_v.at[pl.ds(i0g, 16)], ownS)
        plsc.addupdate(T_v.at[pl.ds(i0g, 16)], ownT)
        for r in range(16):
            rv = jnp.full((16,), r, jnp.int32)
            jv2 = plsc.load_gather(jscr, [iot, rv])
            wv2 = plsc.load_gather(wscr, [iot, rv])
            lv2 = plsc.load_gather(lscr, [iot, rv])
            plsc.addupdate_scatter(S_v, [jv2], wv2, mask=lanemask)
            plsc.addupdate_scatter(T_v, [jv2], lv2, mask=lanemask)
        return 0

    lax.fori_loop(0, _NCH, _chunk, 0)
    pltpu.sync_copy(S_v, pS_h.at[wid])
    pltpu.sync_copy(T_v, pT_h.at[wid])


def _combine_body(accl_ref, ps_ref, pt_ref, out_ref):
    S = jnp.sum(ps_ref[...], axis=0)
    T = jnp.sum(pt_ref[...], axis=0)
    psum = jnp.sum(T / jnp.maximum(S, 1e-12))
    out_ref[...] = jnp.full(
        (1, 1), -(accl_ref[0, 0] + psum) / (_N * _N), jnp.float32)


def kernel(embeddings, data):
    data_bf = data.astype(jnp.bfloat16)
    idx, val = pl.pallas_call(
        _knn_body,
        grid=(_N // _BI1,),
        in_specs=[
            pl.BlockSpec((_BI1, _DH), lambda i: (i, 0)),
            pl.BlockSpec((_N, _DH), lambda i: (0, 0)),
            pl.BlockSpec((_BI1, _DH), lambda i: (i, 0)),
            pl.BlockSpec((_N, _DH), lambda i: (0, 0)),
        ],
        out_specs=[
            pl.BlockSpec((_BI1, 128), lambda i: (i, 0)),
            pl.BlockSpec((_BI1, 128), lambda i: (i, 0)),
        ],
        out_shape=[
            jax.ShapeDtypeStruct((_N, 128), jnp.int32),
            jax.ShapeDtypeStruct((_N, 128), jnp.float32),
        ],
    )(data, data, data_bf, data_bf)

    idx15 = idx[:, :_K]
    val15 = val[:, :_K]
    thr = val[:, _K - 1]
    tidx = idx[:, _K - 1]
    sig = val[:, _K]
    idxc = jnp.transpose(
        jnp.concatenate([jnp.transpose(idx15),
                         jnp.zeros((1, _N), jnp.int32)], axis=0)
        .reshape(16, _NW, 128), (1, 0, 2))
    valc = jnp.transpose(
        jnp.concatenate([jnp.transpose(val15),
                         jnp.zeros((1, _N), jnp.float32)], axis=0)
        .reshape(16, _NW, 128), (1, 0, 2))
    se = jnp.sum(embeddings * embeddings, axis=1)
    eq = jnp.clip(jnp.round(embeddings * _EMB_SCALE),
                  -32768, 32767).astype(jnp.int32)
    embp = jnp.transpose(
        (eq[:, 0::2] & 0xFFFF) | (eq[:, 1::2] << 16))
    lntab = jnp.asarray(_LN_TAB)

    edge = pl.kernel(
        _edge_body,
        mesh=plsc.VectorSubcoreMesh(core_axis_name="c", subcore_axis_name="s"),
        compiler_params=pltpu.CompilerParams(needs_layout_passes=False),
        out_type=[
            jax.ShapeDtypeStruct((_NW, _N), jnp.float32),
            jax.ShapeDtypeStruct((_NW, _N), jnp.float32),
        ],
        scratch_types=[
            pltpu.VMEM((16, _N), jnp.int32),
            pltpu.VMEM((_N,), jnp.float32),
            pltpu.VMEM((_N,), jnp.float32),
            pltpu.VMEM((_N,), jnp.float32),
            pltpu.VMEM((_N,), jnp.int32),
            pltpu.VMEM((_TAB_N + 8,), jnp.float32),
            pltpu.VMEM((16, 128), jnp.float32),
            pltpu.VMEM((16, 128), jnp.int32),
            pltpu.VMEM((_N,), jnp.float32),
            pltpu.VMEM((_N,), jnp.float32),
            pltpu.VMEM((16, 16), jnp.float32),
            pltpu.VMEM((16, 16), jnp.float32),
            pltpu.VMEM((16, 16), jnp.int32),
        ],
    )
    pS, pT = edge(embp, sig, se, thr, tidx, lntab, valc, idxc)

    se_col = se[:, None]
    se_row = se[None, :]
    nb = _N // _BB2
    accl = pl.pallas_call(
        _loss_body,
        grid=(nb, nb // 2 + 1),
        in_specs=[
            pl.BlockSpec((_BB2, _DL), lambda i, t: (i, 0)),
            pl.BlockSpec((_BB2, _DL), lambda i, t: ((i + t) % (_N // _BB2), 0)),
            pl.BlockSpec((_BB2, 1), lambda i, t: (i, 0)),
            pl.BlockSpec((1, _BB2), lambda i, t: (0, (i + t) % (_N // _BB2))),
        ],
        out_specs=pl.BlockSpec((1, 1), lambda i, t: (0, 0)),
        out_shape=jax.ShapeDtypeStruct((1, 1), jnp.float32),
        scratch_shapes=[
            pltpu.SMEM((1, 1), jnp.float32),
        ],
    )(embeddings, embeddings, se_col, se_row)

    out = pl.pallas_call(
        _combine_body,
        grid=(1,),
        in_specs=[
            pl.BlockSpec((1, 1), lambda i: (0, 0)),
            pl.BlockSpec((_NW, _N), lambda i: (0, 0)),
            pl.BlockSpec((_NW, _N), lambda i: (0, 0)),
        ],
        out_specs=pl.BlockSpec((1, 1), lambda i: (0, 0)),
        out_shape=jax.ShapeDtypeStruct((1, 1), jnp.float32),
    )(accl, pS, pT)

    return jnp.reshape(out, ())


# speedup vs baseline: 1.6327x; 1.0448x over previous
import functools

import jax
import jax.numpy as jnp
import numpy as np
from jax import lax
from jax.experimental import pallas as pl
from jax.experimental.pallas import tpu as pltpu
from jax.experimental.pallas import tpu_sc as plsc

_N = 4096
_DH = 512
_DL = 32
_K = 15
_A = 1.0 / 0.1 ** 2
_B = float(np.log(2.0))
_EPS = 1e-7
_LN2 = float(np.log(2.0))

_BI1 = 512
_BI2 = 256
_BJ2 = 512
_BB2 = 512

_LN_TAB = np.log(np.maximum(np.arange(4096 + 8, dtype=np.float64),
                            0.5) / 8.0).astype(np.float32)

_NW = 32
_RW = _N // _NW
_NCH = _RW // 16


def _knn_body(x_ref, y_ref, xb_ref, yb_ref, idx_ref, val_ref):
    i = pl.program_id(0)
    x = x_ref[...]
    y = y_ref[...]
    sx = jnp.sum(x * x, axis=1)[:, None]
    sy = jnp.sum(y * y, axis=1)[None, :]
    g = jax.lax.dot_general(xb_ref[...], yb_ref[...],
                            (((1,), (1,)), ((), ())),
                            preferred_element_type=jnp.float32)
    sq = sx + sy - 2.0 * g
    iglob = i * _BI1 + jax.lax.broadcasted_iota(jnp.int32, (_BI1, _N), 0)
    jlane = jax.lax.broadcasted_iota(jnp.int32, (_BI1, _N), 1)
    kbits = jax.lax.bitcast_convert_type(jnp.maximum(sq, 0.0), jnp.int32)
    key = jnp.where(jlane == iglob, jnp.int32(0x7FFFFFFF),
                    (kbits & jnp.int32(-4096)) | jlane)
    big = jnp.full((_BI1, 128), 0x7FFFFFFF, jnp.int32)
    slices = [key[:, s * 128:(s + 1) * 128] for s in range(32)]
    ms = []
    for _ in range(2):
        cur = big
        for sl in slices:
            hit = jnp.zeros(sl.shape, jnp.bool_)
            for prev in ms:
                hit = jnp.logical_or(hit, sl == prev)
            cur = jnp.minimum(cur, jnp.where(hit, 0x7FFFFFFF, sl))
        ms.append(cur)
    m1, m2 = ms
    cur = m1
    used = jnp.zeros((_BI1, 128), jnp.int32)
    idxs = []
    vals = []
    kmin = jnp.zeros((_BI1,), jnp.int32)
    for _ in range(_K):
        kmin = jnp.min(cur, axis=1)
        idxs.append(kmin & jnp.int32(0xFFF))
        vals.append(jax.lax.bitcast_convert_type(
            kmin & jnp.int32(-4096), jnp.float32))
        onehot = cur == kmin[:, None]
        nxt = jnp.where(used == 0, m2, 0x7FFFFFFF)
        cur = jnp.where(onehot, nxt, cur)
        used = used + onehot.astype(jnp.int32)
    sigma = jnp.sqrt(jnp.maximum(vals[0], 1e-12))
    idx_ref[...] = jnp.stack(idxs + [kmin], axis=0)
    val_ref[...] = jnp.stack(vals + [sigma], axis=0)


def _loss_body(ei_ref, ej_ref, sei_ref, sej_ref, out_ref, accL):
    i = pl.program_id(0)
    t = pl.program_id(1)
    nb = pl.num_programs(0)
    nt = pl.num_programs(1)

    @pl.when(jnp.logical_and(i == 0, t == 0))
    def _():
        accL[0, 0] = 0.0

    @pl.when(jnp.logical_or(t < nt - 1, i < nb // 2))
    def _():
        ei = ei_ref[...]
        ej = ej_ref[...]
        g = jax.lax.dot_general(ei, ej, (((1,), (1,)), ((), ())),
                                preferred_element_type=jnp.float32)
        sq = sei_ref[...] + sej_ref[...] - 2.0 * g
        sql = jnp.maximum(sq, 1e-12)
        u = _A * jnp.exp(_B * jnp.log(sql))
        q = 1.0 / (1.0 + u)
        lnn = jnp.log((1.0 - q) + _EPS)
        wgt = jnp.where(t == 0, 1.0, 2.0)
        accL[0, 0] += wgt * jnp.sum(lnn)

    @pl.when(jnp.logical_and(i == nb - 1, t == nt - 1))
    def _():
        out_ref[...] = jnp.full((1, 1), accL[0, 0], jnp.float32)


_LNA = float(np.log(_A))
_EMB_SCALE = 4096.0
_TAB_N = 4096


def _edge_body(embp_h, sig_h, se_h, thr_h, tidx_h, lntab_h, valc_h, idxc_h,
               pS_h, pT_h,
               embp_v, sig_v, se_v, thr_v, tidx_v, lntab_v, valt_v, idxt_v,
               S_v, T_v, wscr, lscr, jscr):
    nc = 2
    wid = lax.axis_index("s") * nc + lax.axis_index("c")
    pltpu.sync_copy(embp_h, embp_v)
    pltpu.sync_copy(sig_h, sig_v)
    pltpu.sync_copy(se_h, se_v)
    pltpu.sync_copy(thr_h, thr_v)
    pltpu.sync_copy(tidx_h, tidx_v)
    pltpu.sync_copy(lntab_h, lntab_v)
    pltpu.sync_copy(valc_h.at[:, pl.ds(wid * _RW, _RW)], valt_v)
    pltpu.sync_copy(idxc_h.at[:, pl.ds(wid * _RW, _RW)], idxt_v)

    z16 = jnp.zeros((16,), jnp.float32)
    zi16 = jnp.zeros((16,), jnp.int32)

    def _zero(b, _):
        S_v[pl.ds(b * 16, 16)] = z16
        T_v[pl.ds(b * 16, 16)] = z16
        return 0

    lax.fori_loop(0, _N // 16, _zero, 0)
    wscr[_K, :] = z16
    lscr[_K, :] = z16
    jscr[_K, :] = zi16

    iot = lax.iota(jnp.int32, 16)
    lanemask = iot < _K

    def _chunk(c8, _):
        lbase = c8 * 16
        i0g = wid * _RW + lbase
        i_vec = i0g + iot
        sei = se_v[pl.ds(i0g, 16)]
        sigi = sig_v[pl.ds(i0g, 16)]
        inv_sc2 = 1.0 / (_EMB_SCALE * _EMB_SCALE)
        eia = []
        eib = []
        for d in range(16):
            wi = embp_v[d, pl.ds(i0g, 16)]
            eia.append((wi >> 16).astype(jnp.float32))
            eib.append(((wi << 16) >> 16).astype(jnp.float32))
        ownS = z16
        ownT = z16
        for k in range(_K):
            jv = idxt_v[k, pl.ds(lbase, 16)]
            vv = valt_v[k, pl.ds(lbase, 16)]
            sigj = plsc.load_gather(sig_v, [jv])
            sej = plsc.load_gather(se_v, [jv])
            thrj = plsc.load_gather(thr_v, [jv])
            tixj = plsc.load_gather(tidx_v, [jv])
            dot = z16
            for d in range(16):
                wj = plsc.load_gather(embp_v,
                                      [jnp.full((16,), d, jnp.int32), jv])
                aj = (wj >> 16).astype(jnp.float32)
                bj = ((wj << 16) >> 16).astype(jnp.float32)
                dot = dot + eia[d] * aj + eib[d] * bj
            sql = jnp.clip(sei + sej - 2.0 * dot * inv_sc2, 0.25, 511.0)
            sql8 = sql * 8.0
            kidx = sql8.astype(jnp.int32)
            frac = sql8 - kidx.astype(jnp.float32)
            t0 = plsc.load_gather(lntab_v, [kidx])
            t1 = plsc.load_gather(lntab_v, [kidx + 1])
            lnsql = t0 + frac * (t1 - t0)
            u = _A * jnp.exp(_B * lnsql)
            ell = _EPS * u - _LNA - _B * lnsql
            w = jnp.exp(-jnp.maximum(vv, 1e-12) / (sigi * sigj))
            ownS = ownS + w
            ownT = ownT + w * ell
            mut = jnp.logical_or(
                vv < thrj, jnp.logical_and(vv == thrj, i_vec <= tixj))
            nmf = jnp.where(mut, 0.0, 1.0)
            wscr[k, :] = w * nmf
            lscr[k, :] = w * ell * nmf
            jscr[k, :] = jv
        plsc.addupdate(S_v.at[pl.ds(i0g, 16)], ownS)
        plsc.addupdate(T_v.at[pl.ds(i0g, 16)], ownT)
        for r in range(16):
            rv = jnp.full((16,), r, jnp.int32)
            jv2 = plsc.load_gather(jscr, [iot, rv])
            wv2 = plsc.load_gather(wscr, [iot, rv])
            lv2 = plsc.load_gather(lscr, [iot, rv])
            plsc.addupdate_scatter(S_v, [jv2], wv2, mask=lanemask)
            plsc.addupdate_scatter(T_v, [jv2], lv2, mask=lanemask)
        return 0

    lax.fori_loop(0, _NCH, _chunk, 0)
    pltpu.sync_copy(S_v, pS_h.at[wid])
    pltpu.sync_copy(T_v, pT_h.at[wid])


def _combine_body(accl_ref, ps_ref, pt_ref, out_ref):
    S = jnp.sum(ps_ref[...], axis=0)
    T = jnp.sum(pt_ref[...], axis=0)
    psum = jnp.sum(T / jnp.maximum(S, 1e-12))
    out_ref[...] = jnp.full(
        (1, 1), -(accl_ref[0, 0] + psum) / (_N * _N), jnp.float32)


def kernel(embeddings, data):
    data_bf = data.astype(jnp.bfloat16)
    idx, val = pl.pallas_call(
        _knn_body,
        grid=(_N // _BI1,),
        in_specs=[
            pl.BlockSpec((_BI1, _DH), lambda i: (i, 0)),
            pl.BlockSpec((_N, _DH), lambda i: (0, 0)),
            pl.BlockSpec((_BI1, _DH), lambda i: (i, 0)),
            pl.BlockSpec((_N, _DH), lambda i: (0, 0)),
        ],
        out_specs=[
            pl.BlockSpec((16, _BI1), lambda i: (0, i)),
            pl.BlockSpec((16, _BI1), lambda i: (0, i)),
        ],
        out_shape=[
            jax.ShapeDtypeStruct((16, _N), jnp.int32),
            jax.ShapeDtypeStruct((16, _N), jnp.float32),
        ],
    )(data, data, data_bf, data_bf)

    thr = val[_K - 1]
    tidx = idx[_K - 1]
    sig = val[_K]
    se = jnp.sum(embeddings * embeddings, axis=1)
    eq = jnp.clip(jnp.round(embeddings * _EMB_SCALE),
                  -32768, 32767).astype(jnp.int32)
    embp = jnp.transpose(
        (eq[:, 0::2] & 0xFFFF) | (eq[:, 1::2] << 16))
    lntab = jnp.asarray(_LN_TAB)

    edge = pl.kernel(
        _edge_body,
        mesh=plsc.VectorSubcoreMesh(core_axis_name="c", subcore_axis_name="s"),
        compiler_params=pltpu.CompilerParams(needs_layout_passes=False),
        out_type=[
            jax.ShapeDtypeStruct((_NW, _N), jnp.float32),
            jax.ShapeDtypeStruct((_NW, _N), jnp.float32),
        ],
        scratch_types=[
            pltpu.VMEM((16, _N), jnp.int32),
            pltpu.VMEM((_N,), jnp.float32),
            pltpu.VMEM((_N,), jnp.float32),
            pltpu.VMEM((_N,), jnp.float32),
            pltpu.VMEM((_N,), jnp.int32),
            pltpu.VMEM((_TAB_N + 8,), jnp.float32),
            pltpu.VMEM((16, 128), jnp.float32),
            pltpu.VMEM((16, 128), jnp.int32),
            pltpu.VMEM((_N,), jnp.float32),
            pltpu.VMEM((_N,), jnp.float32),
            pltpu.VMEM((16, 16), jnp.float32),
            pltpu.VMEM((16, 16), jnp.float32),
            pltpu.VMEM((16, 16), jnp.int32),
        ],
    )
    pS, pT = edge(embp, sig, se, thr, tidx, lntab, val, idx)

    se_col = se[:, None]
    se_row = se[None, :]
    nb = _N // _BB2
    accl = pl.pallas_call(
        _loss_body,
        grid=(nb, nb // 2 + 1),
        in_specs=[
            pl.BlockSpec((_BB2, _DL), lambda i, t: (i, 0)),
            pl.BlockSpec((_BB2, _DL), lambda i, t: ((i + t) % (_N // _BB2), 0)),
            pl.BlockSpec((_BB2, 1), lambda i, t: (i, 0)),
            pl.BlockSpec((1, _BB2), lambda i, t: (0, (i + t) % (_N // _BB2))),
        ],
        out_specs=pl.BlockSpec((1, 1), lambda i, t: (0, 0)),
        out_shape=jax.ShapeDtypeStruct((1, 1), jnp.float32),
        scratch_shapes=[
            pltpu.SMEM((1, 1), jnp.float32),
        ],
    )(embeddings, embeddings, se_col, se_row)

    out = pl.pallas_call(
        _combine_body,
        grid=(1,),
        in_specs=[
            pl.BlockSpec((1, 1), lambda i: (0, 0)),
            pl.BlockSpec((_NW, _N), lambda i: (0, 0)),
            pl.BlockSpec((_NW, _N), lambda i: (0, 0)),
        ],
        out_specs=pl.BlockSpec((1, 1), lambda i: (0, 0)),
        out_shape=jax.ShapeDtypeStruct((1, 1), jnp.float32),
    )(accl, pS, pT)

    return jnp.reshape(out, ())


# speedup vs baseline: 1.6619x; 1.0179x over previous
import functools

import jax
import jax.numpy as jnp
import numpy as np
from jax import lax
from jax.experimental import pallas as pl
from jax.experimental.pallas import tpu as pltpu
from jax.experimental.pallas import tpu_sc as plsc

_N = 4096
_DH = 512
_DL = 32
_K = 15
_A = 1.0 / 0.1 ** 2
_B = float(np.log(2.0))
_EPS = 1e-7
_LN2 = float(np.log(2.0))

_BI1 = 512
_BI2 = 256
_BJ2 = 512
_BB2 = 512

_LN_TAB = np.log(np.maximum(np.arange(4096 + 8, dtype=np.float64),
                            0.5) / 8.0).astype(np.float32)

_NW = 32
_RW = _N // _NW
_NCH = _RW // 16


def _knn_body(x_ref, y_ref, xb_ref, yb_ref, idx_ref, val_ref, sy_scr):
    i = pl.program_id(0)
    x = x_ref[...]

    @pl.when(i == 0)
    def _():
        y = y_ref[...]
        sy_scr[0:1, :] = jnp.sum(y * y, axis=1)[None, :]

    sx = jnp.sum(x * x, axis=1)[:, None]
    sy = sy_scr[0:1, :]
    g = jax.lax.dot_general(xb_ref[...], yb_ref[...],
                            (((1,), (1,)), ((), ())),
                            preferred_element_type=jnp.float32)
    sq = sx + sy - 2.0 * g
    iglob = i * _BI1 + jax.lax.broadcasted_iota(jnp.int32, (_BI1, _N), 0)
    jlane = jax.lax.broadcasted_iota(jnp.int32, (_BI1, _N), 1)
    kbits = jax.lax.bitcast_convert_type(jnp.maximum(sq, 0.0), jnp.int32)
    key = jnp.where(jlane == iglob, jnp.int32(0x7FFFFFFF),
                    (kbits & jnp.int32(-4096)) | jlane)
    big = jnp.full((_BI1, 128), 0x7FFFFFFF, jnp.int32)
    slices = [key[:, s * 128:(s + 1) * 128] for s in range(32)]
    ms = []
    for _ in range(2):
        cur = big
        for sl in slices:
            hit = jnp.zeros(sl.shape, jnp.bool_)
            for prev in ms:
                hit = jnp.logical_or(hit, sl == prev)
            cur = jnp.minimum(cur, jnp.where(hit, 0x7FFFFFFF, sl))
        ms.append(cur)
    m1, m2 = ms
    cur = m1
    used = jnp.zeros((_BI1, 128), jnp.int32)
    idxs = []
    vals = []
    kmin = jnp.zeros((_BI1,), jnp.int32)
    for _ in range(_K):
        kmin = jnp.min(cur, axis=1)
        idxs.append(kmin & jnp.int32(0xFFF))
        vals.append(jax.lax.bitcast_convert_type(
            kmin & jnp.int32(-4096), jnp.float32))
        onehot = cur == kmin[:, None]
        nxt = jnp.where(used == 0, m2, 0x7FFFFFFF)
        cur = jnp.where(onehot, nxt, cur)
        used = used + onehot.astype(jnp.int32)
    sigma = jnp.sqrt(jnp.maximum(vals[0], 1e-12))
    idx_ref[...] = jnp.stack(idxs + [kmin], axis=0)
    val_ref[...] = jnp.stack(vals + [sigma], axis=0)


def _loss_body(ei_ref, ej_ref, sei_ref, sej_ref, out_ref, accL):
    i = pl.program_id(0)
    t = pl.program_id(1)
    nb = pl.num_programs(0)
    nt = pl.num_programs(1)

    @pl.when(jnp.logical_and(i == 0, t == 0))
    def _():
        accL[0, 0] = 0.0

    @pl.when(jnp.logical_or(t < nt - 1, i < nb // 2))
    def _():
        ei = ei_ref[...]
        ej = ej_ref[...]
        g = jax.lax.dot_general(ei, ej, (((1,), (1,)), ((), ())),
                                preferred_element_type=jnp.float32)
        sq = sei_ref[...] + sej_ref[...] - 2.0 * g
        sql = jnp.maximum(sq, 1e-12)
        u = _A * jnp.exp(_B * jnp.log(sql))
        q = 1.0 / (1.0 + u)
        lnn = jnp.log((1.0 - q) + _EPS)
        wgt = jnp.where(t == 0, 1.0, 2.0)
        accL[0, 0] += wgt * jnp.sum(lnn)

    @pl.when(jnp.logical_and(i == nb - 1, t == nt - 1))
    def _():
        out_ref[...] = jnp.full((1, 1), accL[0, 0], jnp.float32)


_LNA = float(np.log(_A))
_EMB_SCALE = 4096.0
_TAB_N = 4096


def _edge_body(embp_h, sig_h, se_h, thr_h, tidx_h, lntab_h, valc_h, idxc_h,
               pS_h, pT_h,
               embp_v, sig_v, se_v, thr_v, tidx_v, lntab_v, valt_v, idxt_v,
               S_v, T_v, wscr, lscr, jscr):
    nc = 2
    wid = lax.axis_index("s") * nc + lax.axis_index("c")
    pltpu.sync_copy(embp_h, embp_v)
    pltpu.sync_copy(sig_h, sig_v)
    pltpu.sync_copy(se_h, se_v)
    pltpu.sync_copy(thr_h, thr_v)
    pltpu.sync_copy(tidx_h, tidx_v)
    pltpu.sync_copy(lntab_h, lntab_v)
    pltpu.sync_copy(valc_h.at[:, pl.ds(wid * _RW, _RW)], valt_v)
    pltpu.sync_copy(idxc_h.at[:, pl.ds(wid * _RW, _RW)], idxt_v)

    z16 = jnp.zeros((16,), jnp.float32)
    zi16 = jnp.zeros((16,), jnp.int32)

    def _zero(b, _):
        S_v[pl.ds(b * 16, 16)] = z16
        T_v[pl.ds(b * 16, 16)] = z16
        return 0

    lax.fori_loop(0, _N // 16, _zero, 0)
    wscr[_K, :] = z16
    lscr[_K, :] = z16
    jscr[_K, :] = zi16

    iot = lax.iota(jnp.int32, 16)
    lanemask = iot < _K

    def _chunk(c8, _):
        lbase = c8 * 16
        i0g = wid * _RW + lbase
        i_vec = i0g + iot
        sei = se_v[pl.ds(i0g, 16)]
        sigi = sig_v[pl.ds(i0g, 16)]
        inv_sc2 = 1.0 / (_EMB_SCALE * _EMB_SCALE)
        eia = []
        eib = []
        for d in range(16):
            wi = embp_v[d, pl.ds(i0g, 16)]
            eia.append((wi >> 16).astype(jnp.float32))
            eib.append(((wi << 16) >> 16).astype(jnp.float32))
        ownS = z16
        ownT = z16
        for k in range(_K):
            jv = idxt_v[k, pl.ds(lbase, 16)]
            vv = valt_v[k, pl.ds(lbase, 16)]
            sigj = plsc.load_gather(sig_v, [jv])
            sej = plsc.load_gather(se_v, [jv])
            thrj = plsc.load_gather(thr_v, [jv])
            tixj = plsc.load_gather(tidx_v, [jv])
            dot = z16
            for d in range(16):
                wj = plsc.load_gather(embp_v,
                                      [jnp.full((16,), d, jnp.int32), jv])
                aj = (wj >> 16).astype(jnp.float32)
                bj = ((wj << 16) >> 16).astype(jnp.float32)
                dot = dot + eia[d] * aj + eib[d] * bj
            sql = jnp.clip(sei + sej - 2.0 * dot * inv_sc2, 0.25, 511.0)
            sql8 = sql * 8.0
            kidx = sql8.astype(jnp.int32)
            frac = sql8 - kidx.astype(jnp.float32)
            t0 = plsc.load_gather(lntab_v, [kidx])
            t1 = plsc.load_gather(lntab_v, [kidx + 1])
            lnsql = t0 + frac * (t1 - t0)
            u = _A * jnp.exp(_B * lnsql)
            ell = _EPS * u - _LNA - _B * lnsql
            w = jnp.exp(-jnp.maximum(vv, 1e-12) / (sigi * sigj))
            ownS = ownS + w
            ownT = ownT + w * ell
            mut = jnp.logical_or(
                vv < thrj, jnp.logical_and(vv == thrj, i_vec <= tixj))
            nmf = jnp.where(mut, 0.0, 1.0)
            wscr[k, :] = w * nmf
            lscr[k, :] = w * ell * nmf
            jscr[k, :] = jv
        plsc.addupdate(S_v.at[pl.ds(i0g, 16)], ownS)
        plsc.addupdate(T_v.at[pl.ds(i0g, 16)], ownT)
        for r in range(16):
            rv = jnp.full((16,), r, jnp.int32)
            jv2 = plsc.load_gather(jscr, [iot, rv])
            wv2 = plsc.load_gather(wscr, [iot, rv])
            lv2 = plsc.load_gather(lscr, [iot, rv])
            plsc.addupdate_scatter(S_v, [jv2], wv2, mask=lanemask)
            plsc.addupdate_scatter(T_v, [jv2], lv2, mask=lanemask)
        return 0

    lax.fori_loop(0, _NCH, _chunk, 0)
    pltpu.sync_copy(S_v, pS_h.at[wid])
    pltpu.sync_copy(T_v, pT_h.at[wid])


def _combine_body(accl_ref, ps_ref, pt_ref, out_ref):
    S = jnp.sum(ps_ref[...], axis=0)
    T = jnp.sum(pt_ref[...], axis=0)
    psum = jnp.sum(T / jnp.maximum(S, 1e-12))
    out_ref[...] = jnp.full(
        (1, 1), -(accl_ref[0, 0] + psum) / (_N * _N), jnp.float32)


def kernel(embeddings, data):
    data_bf = data.astype(jnp.bfloat16)
    idx, val = pl.pallas_call(
        _knn_body,
        grid=(_N // _BI1,),
        in_specs=[
            pl.BlockSpec((_BI1, _DH), lambda i: (i, 0)),
            pl.BlockSpec((_N, _DH), lambda i: (0, 0)),
            pl.BlockSpec((_BI1, _DH), lambda i: (i, 0)),
            pl.BlockSpec((_N, _DH), lambda i: (0, 0)),
        ],
        out_specs=[
            pl.BlockSpec((16, _BI1), lambda i: (0, i)),
            pl.BlockSpec((16, _BI1), lambda i: (0, i)),
        ],
        out_shape=[
            jax.ShapeDtypeStruct((16, _N), jnp.int32),
            jax.ShapeDtypeStruct((16, _N), jnp.float32),
        ],
        scratch_shapes=[
            pltpu.VMEM((8, _N), jnp.float32),
        ],
    )(data, data, data_bf, data_bf)

    thr = val[_K - 1]
    tidx = idx[_K - 1]
    sig = val[_K]
    se = jnp.sum(embeddings * embeddings, axis=1)
    eq = jnp.clip(jnp.round(embeddings * _EMB_SCALE),
                  -32768, 32767).astype(jnp.int32)
    embp = jnp.transpose(
        (eq[:, 0::2] & 0xFFFF) | (eq[:, 1::2] << 16))
    lntab = jnp.asarray(_LN_TAB)

    edge = pl.kernel(
        _edge_body,
        mesh=plsc.VectorSubcoreMesh(core_axis_name="c", subcore_axis_name="s"),
        compiler_params=pltpu.CompilerParams(needs_layout_passes=False),
        out_type=[
            jax.ShapeDtypeStruct((_NW, _N), jnp.float32),
            jax.ShapeDtypeStruct((_NW, _N), jnp.float32),
        ],
        scratch_types=[
            pltpu.VMEM((16, _N), jnp.int32),
            pltpu.VMEM((_N,), jnp.float32),
            pltpu.VMEM((_N,), jnp.float32),
            pltpu.VMEM((_N,), jnp.float32),
            pltpu.VMEM((_N,), jnp.int32),
            pltpu.VMEM((_TAB_N + 8,), jnp.float32),
            pltpu.VMEM((16, 128), jnp.float32),
            pltpu.VMEM((16, 128), jnp.int32),
            pltpu.VMEM((_N,), jnp.float32),
            pltpu.VMEM((_N,), jnp.float32),
            pltpu.VMEM((16, 16), jnp.float32),
            pltpu.VMEM((16, 16), jnp.float32),
            pltpu.VMEM((16, 16), jnp.int32),
        ],
    )
    pS, pT = edge(embp, sig, se, thr, tidx, lntab, val, idx)

    se_col = se[:, None]
    se_row = se[None, :]
    nb = _N // _BB2
    accl = pl.pallas_call(
        _loss_body,
        grid=(nb, nb // 2 + 1),
        in_specs=[
            pl.BlockSpec((_BB2, _DL), lambda i, t: (i, 0)),
            pl.BlockSpec((_BB2, _DL), lambda i, t: ((i + t) % (_N // _BB2), 0)),
            pl.BlockSpec((_BB2, 1), lambda i, t: (i, 0)),
            pl.BlockSpec((1, _BB2), lambda i, t: (0, (i + t) % (_N // _BB2))),
        ],
        out_specs=pl.BlockSpec((1, 1), lambda i, t: (0, 0)),
        out_shape=jax.ShapeDtypeStruct((1, 1), jnp.float32),
        scratch_shapes=[
            pltpu.SMEM((1, 1), jnp.float32),
        ],
    )(embeddings, embeddings, se_col, se_row)

    out = pl.pallas_call(
        _combine_body,
        grid=(1,),
        in_specs=[
            pl.BlockSpec((1, 1), lambda i: (0, 0)),
            pl.BlockSpec((_NW, _N), lambda i: (0, 0)),
            pl.BlockSpec((_NW, _N), lambda i: (0, 0)),
        ],
        out_specs=pl.BlockSpec((1, 1), lambda i: (0, 0)),
        out_shape=jax.ShapeDtypeStruct((1, 1), jnp.float32),
    )(accl, pS, pT)

    return jnp.reshape(out, ())


# speedup vs baseline: 1.6867x; 1.0149x over previous
import functools

import jax
import jax.numpy as jnp
import numpy as np
from jax import lax
from jax.experimental import pallas as pl
from jax.experimental.pallas import tpu as pltpu
from jax.experimental.pallas import tpu_sc as plsc

_N = 4096
_DH = 512
_DL = 32
_K = 15
_A = 1.0 / 0.1 ** 2
_B = float(np.log(2.0))
_EPS = 1e-7
_LN2 = float(np.log(2.0))

_BI1 = 1024
_BI2 = 256
_BJ2 = 512
_BB2 = 512

_LN_TAB = np.log(np.maximum(np.arange(4096 + 8, dtype=np.float64),
                            0.5) / 8.0).astype(np.float32)

_NW = 32
_RW = _N // _NW
_NCH = _RW // 16


def _knn_body(x_ref, y_ref, xb_ref, yb_ref, idx_ref, val_ref, sy_scr):
    i = pl.program_id(0)
    x = x_ref[...]

    @pl.when(i == 0)
    def _():
        y = y_ref[...]
        sy_scr[0:1, :] = jnp.sum(y * y, axis=1)[None, :]

    sx = jnp.sum(x * x, axis=1)[:, None]
    sy = sy_scr[0:1, :]
    g = jax.lax.dot_general(xb_ref[...], yb_ref[...],
                            (((1,), (1,)), ((), ())),
                            preferred_element_type=jnp.float32)
    sq = sx + sy - 2.0 * g
    iglob = i * _BI1 + jax.lax.broadcasted_iota(jnp.int32, (_BI1, _N), 0)
    jlane = jax.lax.broadcasted_iota(jnp.int32, (_BI1, _N), 1)
    kbits = jax.lax.bitcast_convert_type(jnp.maximum(sq, 0.0), jnp.int32)
    key = jnp.where(jlane == iglob, jnp.int32(0x7FFFFFFF),
                    (kbits & jnp.int32(-4096)) | jlane)
    big = jnp.full((_BI1, 128), 0x7FFFFFFF, jnp.int32)
    slices = [key[:, s * 128:(s + 1) * 128] for s in range(32)]
    ms = []
    for _ in range(2):
        cur = big
        for sl in slices:
            hit = jnp.zeros(sl.shape, jnp.bool_)
            for prev in ms:
                hit = jnp.logical_or(hit, sl == prev)
            cur = jnp.minimum(cur, jnp.where(hit, 0x7FFFFFFF, sl))
        ms.append(cur)
    m1, m2 = ms
    cur = m1
    used = jnp.zeros((_BI1, 128), jnp.int32)
    idxs = []
    vals = []
    kmin = jnp.zeros((_BI1,), jnp.int32)
    for _ in range(_K):
        kmin = jnp.min(cur, axis=1)
        idxs.append(kmin & jnp.int32(0xFFF))
        vals.append(jax.lax.bitcast_convert_type(
            kmin & jnp.int32(-4096), jnp.float32))
        onehot = cur == kmin[:, None]
        nxt = jnp.where(used == 0, m2, 0x7FFFFFFF)
        cur = jnp.where(onehot, nxt, cur)
        used = used + onehot.astype(jnp.int32)
    sigma = jnp.sqrt(jnp.maximum(vals[0], 1e-12))
    idx_ref[...] = jnp.stack(idxs + [kmin], axis=0)
    val_ref[...] = jnp.stack(vals + [sigma], axis=0)


def _loss_body(ei_ref, ej_ref, sei_ref, sej_ref, out_ref, accL):
    i = pl.program_id(0)
    t = pl.program_id(1)
    nb = pl.num_programs(0)
    nt = pl.num_programs(1)

    @pl.when(jnp.logical_and(i == 0, t == 0))
    def _():
        accL[0, 0] = 0.0

    @pl.when(jnp.logical_or(t < nt - 1, i < nb // 2))
    def _():
        ei = ei_ref[...]
        ej = ej_ref[...]
        g = jax.lax.dot_general(ei, ej, (((1,), (1,)), ((), ())),
                                preferred_element_type=jnp.float32)
        sq = sei_ref[...] + sej_ref[...] - 2.0 * g
        sql = jnp.maximum(sq, 1e-12)
        u = _A * jnp.exp(_B * jnp.log(sql))
        q = 1.0 / (1.0 + u)
        lnn = jnp.log((1.0 - q) + _EPS)
        wgt = jnp.where(t == 0, 1.0, 2.0)
        accL[0, 0] += wgt * jnp.sum(lnn)

    @pl.when(jnp.logical_and(i == nb - 1, t == nt - 1))
    def _():
        out_ref[...] = jnp.full((1, 1), accL[0, 0], jnp.float32)


_LNA = float(np.log(_A))
_EMB_SCALE = 4096.0
_TAB_N = 4096


def _edge_body(embp_h, sig_h, se_h, thr_h, tidx_h, lntab_h, valc_h, idxc_h,
               pS_h, pT_h,
               embp_v, sig_v, se_v, thr_v, tidx_v, lntab_v, valt_v, idxt_v,
               S_v, T_v, wscr, lscr, jscr):
    nc = 2
    wid = lax.axis_index("s") * nc + lax.axis_index("c")
    pltpu.sync_copy(embp_h, embp_v)
    pltpu.sync_copy(sig_h, sig_v)
    pltpu.sync_copy(se_h, se_v)
    pltpu.sync_copy(thr_h, thr_v)
    pltpu.sync_copy(tidx_h, tidx_v)
    pltpu.sync_copy(lntab_h, lntab_v)
    pltpu.sync_copy(valc_h.at[:, pl.ds(wid * _RW, _RW)], valt_v)
    pltpu.sync_copy(idxc_h.at[:, pl.ds(wid * _RW, _RW)], idxt_v)

    z16 = jnp.zeros((16,), jnp.float32)
    zi16 = jnp.zeros((16,), jnp.int32)

    def _zero(b, _):
        S_v[pl.ds(b * 16, 16)] = z16
        T_v[pl.ds(b * 16, 16)] = z16
        return 0

    lax.fori_loop(0, _N // 16, _zero, 0)
    wscr[_K, :] = z16
    lscr[_K, :] = z16
    jscr[_K, :] = zi16

    iot = lax.iota(jnp.int32, 16)
    lanemask = iot < _K

    def _chunk(c8, _):
        lbase = c8 * 16
        i0g = wid * _RW + lbase
        i_vec = i0g + iot
        sei = se_v[pl.ds(i0g, 16)]
        sigi = sig_v[pl.ds(i0g, 16)]
        inv_sc2 = 1.0 / (_EMB_SCALE * _EMB_SCALE)
        eia = []
        eib = []
        for d in range(16):
            wi = embp_v[d, pl.ds(i0g, 16)]
            eia.append((wi >> 16).astype(jnp.float32))
            eib.append(((wi << 16) >> 16).astype(jnp.float32))
        ownS = z16
        ownT = z16
        for k in range(_K):
            jv = idxt_v[k, pl.ds(lbase, 16)]
            vv = valt_v[k, pl.ds(lbase, 16)]
            sigj = plsc.load_gather(sig_v, [jv])
            sej = plsc.load_gather(se_v, [jv])
            thrj = plsc.load_gather(thr_v, [jv])
            tixj = plsc.load_gather(tidx_v, [jv])
            dot = z16
            for d in range(16):
                wj = plsc.load_gather(embp_v,
                                      [jnp.full((16,), d, jnp.int32), jv])
                aj = (wj >> 16).astype(jnp.float32)
                bj = ((wj << 16) >> 16).astype(jnp.float32)
                dot = dot + eia[d] * aj + eib[d] * bj
            sql = jnp.clip(sei + sej - 2.0 * dot * inv_sc2, 0.25, 511.0)
            sql8 = sql * 8.0
            kidx = sql8.astype(jnp.int32)
            frac = sql8 - kidx.astype(jnp.float32)
            t0 = plsc.load_gather(lntab_v, [kidx])
            t1 = plsc.load_gather(lntab_v, [kidx + 1])
            lnsql = t0 + frac * (t1 - t0)
            u = _A * jnp.exp(_B * lnsql)
            ell = _EPS * u - _LNA - _B * lnsql
            w = jnp.exp(-jnp.maximum(vv, 1e-12) / (sigi * sigj))
            ownS = ownS + w
            ownT = ownT + w * ell
            mut = jnp.logical_or(
                vv < thrj, jnp.logical_and(vv == thrj, i_vec <= tixj))
            nmf = jnp.where(mut, 0.0, 1.0)
            wscr[k, :] = w * nmf
            lscr[k, :] = w * ell * nmf
            jscr[k, :] = jv
        plsc.addupdate(S_v.at[pl.ds(i0g, 16)], ownS)
        plsc.addupdate(T_v.at[pl.ds(i0g, 16)], ownT)
        for r in range(16):
            rv = jnp.full((16,), r, jnp.int32)
            jv2 = plsc.load_gather(jscr, [iot, rv])
            wv2 = plsc.load_gather(wscr, [iot, rv])
            lv2 = plsc.load_gather(lscr, [iot, rv])
            plsc.addupdate_scatter(S_v, [jv2], wv2, mask=lanemask)
            plsc.addupdate_scatter(T_v, [jv2], lv2, mask=lanemask)
        return 0

    lax.fori_loop(0, _NCH, _chunk, 0)
    pltpu.sync_copy(S_v, pS_h.at[wid])
    pltpu.sync_copy(T_v, pT_h.at[wid])


def _combine_body(accl_ref, ps_ref, pt_ref, out_ref):
    S = jnp.sum(ps_ref[...], axis=0)
    T = jnp.sum(pt_ref[...], axis=0)
    psum = jnp.sum(T / jnp.maximum(S, 1e-12))
    out_ref[...] = jnp.full(
        (1, 1), -(accl_ref[0, 0] + psum) / (_N * _N), jnp.float32)


def kernel(embeddings, data):
    data_bf = data.astype(jnp.bfloat16)
    idx, val = pl.pallas_call(
        _knn_body,
        grid=(_N // _BI1,),
        in_specs=[
            pl.BlockSpec((_BI1, _DH), lambda i: (i, 0)),
            pl.BlockSpec((_N, _DH), lambda i: (0, 0)),
            pl.BlockSpec((_BI1, _DH), lambda i: (i, 0)),
            pl.BlockSpec((_N, _DH), lambda i: (0, 0)),
        ],
        out_specs=[
            pl.BlockSpec((16, _BI1), lambda i: (0, i)),
            pl.BlockSpec((16, _BI1), lambda i: (0, i)),
        ],
        out_shape=[
            jax.ShapeDtypeStruct((16, _N), jnp.int32),
            jax.ShapeDtypeStruct((16, _N), jnp.float32),
        ],
        scratch_shapes=[
            pltpu.VMEM((8, _N), jnp.float32),
        ],
    )(data, data, data_bf, data_bf)

    thr = val[_K - 1]
    tidx = idx[_K - 1]
    sig = val[_K]
    se = jnp.sum(embeddings * embeddings, axis=1)
    eq = jnp.clip(jnp.round(embeddings * _EMB_SCALE),
                  -32768, 32767).astype(jnp.int32)
    embp = jnp.transpose(
        (eq[:, 0::2] & 0xFFFF) | (eq[:, 1::2] << 16))
    lntab = jnp.asarray(_LN_TAB)

    edge = pl.kernel(
        _edge_body,
        mesh=plsc.VectorSubcoreMesh(core_axis_name="c", subcore_axis_name="s"),
        compiler_params=pltpu.CompilerParams(needs_layout_passes=False),
        out_type=[
            jax.ShapeDtypeStruct((_NW, _N), jnp.float32),
            jax.ShapeDtypeStruct((_NW, _N), jnp.float32),
        ],
        scratch_types=[
            pltpu.VMEM((16, _N), jnp.int32),
            pltpu.VMEM((_N,), jnp.float32),
            pltpu.VMEM((_N,), jnp.float32),
            pltpu.VMEM((_N,), jnp.float32),
            pltpu.VMEM((_N,), jnp.int32),
            pltpu.VMEM((_TAB_N + 8,), jnp.float32),
            pltpu.VMEM((16, 128), jnp.float32),
            pltpu.VMEM((16, 128), jnp.int32),
            pltpu.VMEM((_N,), jnp.float32),
            pltpu.VMEM((_N,), jnp.float32),
            pltpu.VMEM((16, 16), jnp.float32),
            pltpu.VMEM((16, 16), jnp.float32),
            pltpu.VMEM((16, 16), jnp.int32),
        ],
    )
    pS, pT = edge(embp, sig, se, thr, tidx, lntab, val, idx)

    se_col = se[:, None]
    se_row = se[None, :]
    nb = _N // _BB2
    accl = pl.pallas_call(
        _loss_body,
        grid=(nb, nb // 2 + 1),
        in_specs=[
            pl.BlockSpec((_BB2, _DL), lambda i, t: (i, 0)),
            pl.BlockSpec((_BB2, _DL), lambda i, t: ((i + t) % (_N // _BB2), 0)),
            pl.BlockSpec((_BB2, 1), lambda i, t: (i, 0)),
            pl.BlockSpec((1, _BB2), lambda i, t: (0, (i + t) % (_N // _BB2))),
        ],
        out_specs=pl.BlockSpec((1, 1), lambda i, t: (0, 0)),
        out_shape=jax.ShapeDtypeStruct((1, 1), jnp.float32),
        scratch_shapes=[
            pltpu.SMEM((1, 1), jnp.float32),
        ],
    )(embeddings, embeddings, se_col, se_row)

    out = pl.pallas_call(
        _combine_body,
        grid=(1,),
        in_specs=[
            pl.BlockSpec((1, 1), lambda i: (0, 0)),
            pl.BlockSpec((_NW, _N), lambda i: (0, 0)),
            pl.BlockSpec((_NW, _N), lambda i: (0, 0)),
        ],
        out_specs=pl.BlockSpec((1, 1), lambda i: (0, 0)),
        out_shape=jax.ShapeDtypeStruct((1, 1), jnp.float32),
    )(accl, pS, pT)

    return jnp.reshape(out, ())


# speedup vs baseline: 1.7016x; 1.0089x over previous
import functools

import jax
import jax.numpy as jnp
import numpy as np
from jax import lax
from jax.experimental import pallas as pl
from jax.experimental.pallas import tpu as pltpu
from jax.experimental.pallas import tpu_sc as plsc

_N = 4096
_DH = 512
_DL = 32
_K = 15
_A = 1.0 / 0.1 ** 2
_B = float(np.log(2.0))
_EPS = 1e-7
_LN2 = float(np.log(2.0))

_BI1 = 1024
_BI2 = 256
_BJ2 = 512
_BB2 = 1024

_LN_TAB = np.log(np.maximum(np.arange(4096 + 8, dtype=np.float64),
                            0.5) / 8.0).astype(np.float32)

_NW = 32
_RW = _N // _NW
_NCH = _RW // 16


def _knn_body(x_ref, y_ref, xb_ref, yb_ref, idx_ref, val_ref, sy_scr):
    i = pl.program_id(0)
    x = x_ref[...]

    @pl.when(i == 0)
    def _():
        y = y_ref[...]
        sy_scr[0:1, :] = jnp.sum(y * y, axis=1)[None, :]

    sx = jnp.sum(x * x, axis=1)[:, None]
    sy = sy_scr[0:1, :]
    g = jax.lax.dot_general(xb_ref[...], yb_ref[...],
                            (((1,), (1,)), ((), ())),
                            preferred_element_type=jnp.float32)
    sq = sx + sy - 2.0 * g
    iglob = i * _BI1 + jax.lax.broadcasted_iota(jnp.int32, (_BI1, _N), 0)
    jlane = jax.lax.broadcasted_iota(jnp.int32, (_BI1, _N), 1)
    kbits = jax.lax.bitcast_convert_type(jnp.maximum(sq, 0.0), jnp.int32)
    key = jnp.where(jlane == iglob, jnp.int32(0x7FFFFFFF),
                    (kbits & jnp.int32(-4096)) | jlane)
    big = jnp.full((_BI1, 128), 0x7FFFFFFF, jnp.int32)
    slices = [key[:, s * 128:(s + 1) * 128] for s in range(32)]
    ms = []
    for _ in range(2):
        cur = big
        for sl in slices:
            hit = jnp.zeros(sl.shape, jnp.bool_)
            for prev in ms:
                hit = jnp.logical_or(hit, sl == prev)
            cur = jnp.minimum(cur, jnp.where(hit, 0x7FFFFFFF, sl))
        ms.append(cur)
    m1, m2 = ms
    cur = m1
    used = jnp.zeros((_BI1, 128), jnp.int32)
    idxs = []
    vals = []
    kmin = jnp.zeros((_BI1,), jnp.int32)
    for _ in range(_K):
        kmin = jnp.min(cur, axis=1)
        idxs.append(kmin & jnp.int32(0xFFF))
        vals.append(jax.lax.bitcast_convert_type(
            kmin & jnp.int32(-4096), jnp.float32))
        onehot = cur == kmin[:, None]
        nxt = jnp.where(used == 0, m2, 0x7FFFFFFF)
        cur = jnp.where(onehot, nxt, cur)
        used = used + onehot.astype(jnp.int32)
    sigma = jnp.sqrt(jnp.maximum(vals[0], 1e-12))
    idx_ref[...] = jnp.stack(idxs + [kmin], axis=0)
    val_ref[...] = jnp.stack(vals + [sigma], axis=0)


def _loss_body(ei_ref, ej_ref, sei_ref, sej_ref, out_ref, accL):
    i = pl.program_id(0)
    t = pl.program_id(1)
    nb = pl.num_programs(0)
    nt = pl.num_programs(1)

    @pl.when(jnp.logical_and(i == 0, t == 0))
    def _():
        accL[0, 0] = 0.0

    @pl.when(jnp.logical_or(t < nt - 1, i < nb // 2))
    def _():
        ei = ei_ref[...]
        ej = ej_ref[...]
        g = jax.lax.dot_general(ei, ej, (((1,), (1,)), ((), ())),
                                preferred_element_type=jnp.float32)
        sq = sei_ref[...] + sej_ref[...] - 2.0 * g
        sql = jnp.maximum(sq, 1e-12)
        u = _A * jnp.exp(_B * jnp.log(sql))
        q = 1.0 / (1.0 + u)
        lnn = jnp.log((1.0 - q) + _EPS)
        wgt = jnp.where(t == 0, 1.0, 2.0)
        accL[0, 0] += wgt * jnp.sum(lnn)

    @pl.when(jnp.logical_and(i == nb - 1, t == nt - 1))
    def _():
        out_ref[...] = jnp.full((1, 1), accL[0, 0], jnp.float32)


_LNA = float(np.log(_A))
_EMB_SCALE = 4096.0
_TAB_N = 4096


def _edge_body(embp_h, sig_h, se_h, thr_h, tidx_h, lntab_h, valc_h, idxc_h,
               pS_h, pT_h,
               embp_v, sig_v, se_v, thr_v, tidx_v, lntab_v, valt_v, idxt_v,
               S_v, T_v, wscr, lscr, jscr):
    nc = 2
    wid = lax.axis_index("s") * nc + lax.axis_index("c")
    pltpu.sync_copy(embp_h, embp_v)
    pltpu.sync_copy(sig_h, sig_v)
    pltpu.sync_copy(se_h, se_v)
    pltpu.sync_copy(thr_h, thr_v)
    pltpu.sync_copy(tidx_h, tidx_v)
    pltpu.sync_copy(lntab_h, lntab_v)
    pltpu.sync_copy(valc_h.at[:, pl.ds(wid * _RW, _RW)], valt_v)
    pltpu.sync_copy(idxc_h.at[:, pl.ds(wid * _RW, _RW)], idxt_v)

    z16 = jnp.zeros((16,), jnp.float32)
    zi16 = jnp.zeros((16,), jnp.int32)

    def _zero(b, _):
        S_v[pl.ds(b * 16, 16)] = z16
        T_v[pl.ds(b * 16, 16)] = z16
        return 0

    lax.fori_loop(0, _N // 16, _zero, 0)
    wscr[_K, :] = z16
    lscr[_K, :] = z16
    jscr[_K, :] = zi16

    iot = lax.iota(jnp.int32, 16)
    lanemask = iot < _K

    def _chunk(c8, _):
        lbase = c8 * 16
        i0g = wid * _RW + lbase
        i_vec = i0g + iot
        sei = se_v[pl.ds(i0g, 16)]
        sigi = sig_v[pl.ds(i0g, 16)]
        inv_sc2 = 1.0 / (_EMB_SCALE * _EMB_SCALE)
        eia = []
        eib = []
        for d in range(16):
            wi = embp_v[d, pl.ds(i0g, 16)]
            eia.append((wi >> 16).astype(jnp.float32))
            eib.append(((wi << 16) >> 16).astype(jnp.float32))
        ownS = z16
        ownT = z16
        for k in range(_K):
            jv = idxt_v[k, pl.ds(lbase, 16)]
            vv = valt_v[k, pl.ds(lbase, 16)]
            sigj = plsc.load_gather(sig_v, [jv])
            sej = plsc.load_gather(se_v, [jv])
            thrj = plsc.load_gather(thr_v, [jv])
            tixj = plsc.load_gather(tidx_v, [jv])
            dot = z16
            for d in range(16):
                wj = plsc.load_gather(embp_v,
                                      [jnp.full((16,), d, jnp.int32), jv])
                aj = (wj >> 16).astype(jnp.float32)
                bj = ((wj << 16) >> 16).astype(jnp.float32)
                dot = dot + eia[d] * aj + eib[d] * bj
            sql = jnp.clip(sei + sej - 2.0 * dot * inv_sc2, 0.25, 511.0)
            sql8 = sql * 8.0
            kidx = sql8.astype(jnp.int32)
            frac = sql8 - kidx.astype(jnp.float32)
            t0 = plsc.load_gather(lntab_v, [kidx])
            t1 = plsc.load_gather(lntab_v, [kidx + 1])
            lnsql = t0 + frac * (t1 - t0)
            u = _A * jnp.exp(_B * lnsql)
            ell = _EPS * u - _LNA - _B * lnsql
            w = jnp.exp(-jnp.maximum(vv, 1e-12) / (sigi * sigj))
            ownS = ownS + w
            ownT = ownT + w * ell
            mut = jnp.logical_or(
                vv < thrj, jnp.logical_and(vv == thrj, i_vec <= tixj))
            nmf = jnp.where(mut, 0.0, 1.0)
            wscr[k, :] = w * nmf
            lscr[k, :] = w * ell * nmf
            jscr[k, :] = jv
        plsc.addupdate(S_v.at[pl.ds(i0g, 16)], ownS)
        plsc.addupdate(T_v.at[pl.ds(i0g, 16)], ownT)
        for r in range(16):
            rv = jnp.full((16,), r, jnp.int32)
            jv2 = plsc.load_gather(jscr, [iot, rv])
            wv2 = plsc.load_gather(wscr, [iot, rv])
            lv2 = plsc.load_gather(lscr, [iot, rv])
            plsc.addupdate_scatter(S_v, [jv2], wv2, mask=lanemask)
            plsc.addupdate_scatter(T_v, [jv2], lv2, mask=lanemask)
        return 0

    lax.fori_loop(0, _NCH, _chunk, 0)
    pltpu.sync_copy(S_v, pS_h.at[wid])
    pltpu.sync_copy(T_v, pT_h.at[wid])


def _combine_body(accl_ref, ps_ref, pt_ref, out_ref):
    S = jnp.sum(ps_ref[...], axis=0)
    T = jnp.sum(pt_ref[...], axis=0)
    psum = jnp.sum(T / jnp.maximum(S, 1e-12))
    out_ref[...] = jnp.full(
        (1, 1), -(accl_ref[0, 0] + psum) / (_N * _N), jnp.float32)


def kernel(embeddings, data):
    data_bf = data.astype(jnp.bfloat16)
    idx, val = pl.pallas_call(
        _knn_body,
        grid=(_N // _BI1,),
        in_specs=[
            pl.BlockSpec((_BI1, _DH), lambda i: (i, 0)),
            pl.BlockSpec((_N, _DH), lambda i: (0, 0)),
            pl.BlockSpec((_BI1, _DH), lambda i: (i, 0)),
            pl.BlockSpec((_N, _DH), lambda i: (0, 0)),
        ],
        out_specs=[
            pl.BlockSpec((16, _BI1), lambda i: (0, i)),
            pl.BlockSpec((16, _BI1), lambda i: (0, i)),
        ],
        out_shape=[
            jax.ShapeDtypeStruct((16, _N), jnp.int32),
            jax.ShapeDtypeStruct((16, _N), jnp.float32),
        ],
        scratch_shapes=[
            pltpu.VMEM((8, _N), jnp.float32),
        ],
    )(data, data, data_bf, data_bf)

    thr = val[_K - 1]
    tidx = idx[_K - 1]
    sig = val[_K]
    se = jnp.sum(embeddings * embeddings, axis=1)
    eq = jnp.clip(jnp.round(embeddings * _EMB_SCALE),
                  -32768, 32767).astype(jnp.int32)
    embp = jnp.transpose(
        (eq[:, 0::2] & 0xFFFF) | (eq[:, 1::2] << 16))
    lntab = jnp.asarray(_LN_TAB)

    edge = pl.kernel(
        _edge_body,
        mesh=plsc.VectorSubcoreMesh(core_axis_name="c", subcore_axis_name="s"),
        compiler_params=pltpu.CompilerParams(needs_layout_passes=False),
        out_type=[
            jax.ShapeDtypeStruct((_NW, _N), jnp.float32),
            jax.ShapeDtypeStruct((_NW, _N), jnp.float32),
        ],
        scratch_types=[
            pltpu.VMEM((16, _N), jnp.int32),
            pltpu.VMEM((_N,), jnp.float32),
            pltpu.VMEM((_N,), jnp.float32),
            pltpu.VMEM((_N,), jnp.float32),
            pltpu.VMEM((_N,), jnp.int32),
            pltpu.VMEM((_TAB_N + 8,), jnp.float32),
            pltpu.VMEM((16, 128), jnp.float32),
            pltpu.VMEM((16, 128), jnp.int32),
            pltpu.VMEM((_N,), jnp.float32),
            pltpu.VMEM((_N,), jnp.float32),
            pltpu.VMEM((16, 16), jnp.float32),
            pltpu.VMEM((16, 16), jnp.float32),
            pltpu.VMEM((16, 16), jnp.int32),
        ],
    )
    pS, pT = edge(embp, sig, se, thr, tidx, lntab, val, idx)

    se_col = se[:, None]
    se_row = se[None, :]
    nb = _N // _BB2
    accl = pl.pallas_call(
        _loss_body,
        grid=(nb, nb // 2 + 1),
        in_specs=[
            pl.BlockSpec((_BB2, _DL), lambda i, t: (i, 0)),
            pl.BlockSpec((_BB2, _DL), lambda i, t: ((i + t) % (_N // _BB2), 0)),
            pl.BlockSpec((_BB2, 1), lambda i, t: (i, 0)),
            pl.BlockSpec((1, _BB2), lambda i, t: (0, (i + t) % (_N // _BB2))),
        ],
        out_specs=pl.BlockSpec((1, 1), lambda i, t: (0, 0)),
        out_shape=jax.ShapeDtypeStruct((1, 1), jnp.float32),
        scratch_shapes=[
            pltpu.SMEM((1, 1), jnp.float32),
        ],
    )(embeddings, embeddings, se_col, se_row)

    out = pl.pallas_call(
        _combine_body,
        grid=(1,),
        in_specs=[
            pl.BlockSpec((1, 1), lambda i: (0, 0)),
            pl.BlockSpec((_NW, _N), lambda i: (0, 0)),
            pl.BlockSpec((_NW, _N), lambda i: (0, 0)),
        ],
        out_specs=pl.BlockSpec((1, 1), lambda i: (0, 0)),
        out_shape=jax.ShapeDtypeStruct((1, 1), jnp.float32),
    )(accl, pS, pT)

    return jnp.reshape(out, ())
